# Initial kernel scaffold; baseline (speedup 1.0000x reference)
import jax
import jax.numpy as jnp
from jax.experimental import pallas as pl


def kernel(x, edge_index, batch, emb, lin1_w, lin1_b, lin2_w, lin2_b):
    raise NotImplementedError("write your pallas kernel here")


# speedup vs baseline: 12.3362x; 12.3362x over previous
import functools

import jax
import jax.numpy as jnp
from jax import lax
from jax.experimental import pallas as pl
from jax.experimental.pallas import tpu as pltpu
from jax.experimental.pallas import tpu_sc as plsc

N = 10000
NPAD = 10240
ROWS = NPAD // 128
NGRAPH = 128
NEDGE = 320000

SC_CORES = 2
SC_SUBCORES = 16
SC_WORKERS = SC_CORES * SC_SUBCORES
EDGE_ROWS_PER_W = 80
EDGE_ROWS = SC_WORKERS * EDGE_ROWS_PER_W
EPAD = EDGE_ROWS * 128

MIX_M = 0x45D9F3B
K_NEIGH = 0x9E3779B1 - (1 << 32)
K_POS = 0x85EBCA6B - (1 << 32)
SIGN = -0x80000000
IMAX = 0x7FFFFFFF


def _mix_i32(a):
    m = jnp.int32(MIX_M)
    a = (a ^ lax.shift_right_logical(a, 16)) * m
    a = (a ^ lax.shift_right_logical(a, 16)) * m
    return a ^ lax.shift_right_logical(a, 16)


def _argmax_body(x_ref, o_ref):
    xb = x_ref[...]
    mx = jnp.max(xb, axis=1, keepdims=True)
    it = lax.broadcasted_iota(jnp.int32, xb.shape, 1)
    cand = jnp.where(xb == mx, it, jnp.int32(128))
    idx = jnp.min(cand, axis=1)
    o_ref[...] = _mix_i32(jnp.reshape(idx, (8, 128)))


def _argmax_mix(x):
    return pl.pallas_call(
        _argmax_body,
        grid=(ROWS // 8,),
        in_specs=[pl.BlockSpec((1024, 128), lambda i: (i, 0))],
        out_specs=pl.BlockSpec((8, 128), lambda i: (i, 0)),
        out_shape=jax.ShapeDtypeStruct((ROWS, 128), jnp.int32),
    )(x)


def _seg_body(h_hbm, src_hbm, dst_hbm, zeros_hbm, out_hbm,
              src_v, dst_v, gath_v, acc_sh, sem):
    cid = lax.axis_index("c")
    sid = lax.axis_index("s")
    wid = cid * SC_SUBCORES + sid
    base = wid * EDGE_ROWS_PER_W
    pltpu.sync_copy(src_hbm.at[pl.ds(base, EDGE_ROWS_PER_W)], src_v)
    pltpu.sync_copy(dst_hbm.at[pl.ds(base, EDGE_ROWS_PER_W)], dst_v)

    @pl.when(sid == 0)
    def _():
        pltpu.sync_copy(zeros_hbm, acc_sh)

    plsc.subcore_barrier()

    def edge_batch(t, carry):
        descs = [
            pltpu.async_copy(h_hbm.at[src_v.at[t * 8 + b]],
                             gath_v.at[t * 8 + b], sem)
            for b in range(8)
        ]
        for d in descs:
            d.wait()
        for b in range(8):
            pltpu.sync_copy(gath_v.at[t * 8 + b],
                            acc_sh.at[dst_v.at[t * 8 + b]], add=True)
        return carry

    lax.fori_loop(0, EDGE_ROWS_PER_W // 8, edge_batch, 0)
    plsc.subcore_barrier()

    @pl.when(sid == 0)
    def _():
        pltpu.sync_copy(acc_sh, out_hbm.at[cid])


@functools.cache
def _seg_sum_sc_fn():
    return functools.partial(
        pl.kernel,
        out_type=jax.ShapeDtypeStruct((SC_CORES, NPAD), jnp.int32),
        mesh=plsc.VectorSubcoreMesh(
            core_axis_name="c", subcore_axis_name="s",
            num_cores=SC_CORES, num_subcores=SC_SUBCORES),
        scratch_types=[
            pltpu.VMEM((EDGE_ROWS_PER_W, 128), jnp.int32),
            pltpu.VMEM((EDGE_ROWS_PER_W, 128), jnp.int32),
            pltpu.VMEM((EDGE_ROWS_PER_W, 128), jnp.int32),
            pltpu.VMEM_SHARED((NPAD,), jnp.int32),
            pltpu.SemaphoreType.DMA,
        ],
    )(_seg_body)


def _seg_sum_sc(h_flat, src, dst, zeros):
    return _seg_sum_sc_fn()(h_flat, src, dst, zeros)


def _comb_body(h_ref, na_ref, nb_ref, o_ref):
    h = h_ref[...]
    neigh = na_ref[...] + nb_ref[...]
    val = _mix_i32(h + jnp.int32(K_NEIGH) * neigh)
    gidx = (lax.broadcasted_iota(jnp.int32, (ROWS, 128), 0) * 128
            + lax.broadcasted_iota(jnp.int32, (ROWS, 128), 1))
    o_ref[...] = jnp.where(gidx >= N, jnp.int32(IMAX), val ^ jnp.int32(SIGN))


def _comb(h2d, na, nb):
    return pl.pallas_call(
        _comb_body,
        in_specs=[pl.BlockSpec((ROWS, 128), lambda: (0, 0))] * 3,
        out_specs=pl.BlockSpec((ROWS, 128), lambda: (0, 0)),
        out_shape=jax.ShapeDtypeStruct((ROWS, 128), jnp.int32),
    )(h2d, na, nb)


def _first_body(vfull_ref, vblk_ref, o_ref):
    pid = pl.program_id(0)
    vb = vblk_ref[...]
    for js in range(8):
        a = jnp.reshape(vb[js:js + 1, :], (128, 1))
        jidx = (lax.broadcasted_iota(jnp.int32, (128, 1), 0)
                + (pid * 8 + js) * 128)

        def body(k, acc):
            ch = vfull_ref[pl.ds(k, 1), :]
            kidx = k * 128 + lax.broadcasted_iota(jnp.int32, (1, 128), 1)
            hit = (a == ch) & (kidx < jidx)
            return acc + jnp.where(hit, jnp.int32(1), jnp.int32(0))

        acc = lax.fori_loop(0, ROWS, body, jnp.zeros((128, 128), jnp.int32))
        cnt = jnp.sum(acc, axis=1)
        o_ref[js:js + 1, :] = jnp.reshape(
            jnp.where(cnt == 0, jnp.int32(1), jnp.int32(0)), (1, 128))


def _first_occ(valb):
    return pl.pallas_call(
        _first_body,
        grid=(ROWS // 8,),
        in_specs=[pl.BlockSpec((ROWS, 128), lambda i: (0, 0)),
                  pl.BlockSpec((8, 128), lambda i: (i, 0))],
        out_specs=pl.BlockSpec((8, 128), lambda i: (i, 0)),
        out_shape=jax.ShapeDtypeStruct((ROWS, 128), jnp.int32),
    )(valb, valb)


def _rank_body(mix_out, vfull_ref, f_ref, vblk_ref, o_ref):
    vb = vblk_ref[...]
    for js in range(8):
        a = jnp.reshape(vb[js:js + 1, :], (128, 1))

        def body(k, acc):
            ch = vfull_ref[pl.ds(k, 1), :]
            fch = f_ref[pl.ds(k, 1), :]
            hit = (ch < a) & (fch != 0)
            return acc + jnp.where(hit, jnp.int32(1), jnp.int32(0))

        acc = lax.fori_loop(0, ROWS, body, jnp.zeros((128, 128), jnp.int32))
        rank = jnp.sum(acc, axis=1)
        if mix_out:
            rank = _mix_i32(rank)
        o_ref[js:js + 1, :] = jnp.reshape(rank, (1, 128))


def _rank(valb, f, mix_out):
    return pl.pallas_call(
        functools.partial(_rank_body, mix_out),
        grid=(ROWS // 8,),
        in_specs=[pl.BlockSpec((ROWS, 128), lambda i: (0, 0)),
                  pl.BlockSpec((ROWS, 128), lambda i: (0, 0)),
                  pl.BlockSpec((8, 128), lambda i: (i, 0))],
        out_specs=pl.BlockSpec((8, 128), lambda i: (i, 0)),
        out_shape=jax.ShapeDtypeStruct((ROWS, 128), jnp.int32),
    )(valb, f, valb)


def _final_body(colors_ref, batch_ref, emb_ref, w1_ref, b1_ref,
                w2_ref, b2_ref, o_ref):
    g_row = lax.broadcasted_iota(jnp.int32, (1, 128), 1)
    s_col = lax.broadcasted_iota(jnp.int32, (128, 1), 0)

    def starts_body(r, acc):
        bt = jnp.reshape(batch_ref[pl.ds(r, 1), :], (128, 1))
        return acc + jnp.where(bt < g_row, jnp.int32(1), jnp.int32(0))

    sacc = lax.fori_loop(0, ROWS, starts_body,
                         jnp.zeros((128, 128), jnp.int32))
    starts_row = jnp.reshape(jnp.sum(sacc, axis=0), (1, 128))

    def gsig_body(r, gacc):
        bt = jnp.reshape(batch_ref[pl.ds(r, 1), :], (128, 1))
        ct = jnp.reshape(colors_ref[pl.ds(r, 1), :], (128, 1))
        eqg = bt == g_row
        st_e = jnp.sum(jnp.where(eqg, starts_row, jnp.int32(0)),
                       axis=1, keepdims=True)
        flat = r * 128 + s_col
        sig = _mix_i32(ct + jnp.int32(K_POS) * (flat - st_e))
        return gacc + jnp.where(eqg, sig, jnp.int32(0))

    gacc = lax.fori_loop(0, ROWS, gsig_body,
                         jnp.zeros((128, 128), jnp.int32))
    gsig_row = jnp.reshape(jnp.sum(gacc, axis=0), (1, 128))

    gb = gsig_row ^ jnp.int32(SIGN)
    gt = jnp.reshape(gb, (128, 1))
    dup = jnp.sum(jnp.where((gt == gb) & (g_row < s_col),
                            jnp.int32(1), jnp.int32(0)), axis=1)
    f_row = jnp.reshape(jnp.where(dup == 0, jnp.int32(1), jnp.int32(0)),
                        (1, 128))
    rank = jnp.sum(jnp.where((gb < gt) & (f_row != 0),
                             jnp.int32(1), jnp.int32(0)),
                   axis=1)
    oh = (jnp.reshape(rank, (128, 1)) == g_row).astype(jnp.float32)
    gx = jnp.dot(oh, emb_ref[...], preferred_element_type=jnp.float32)
    h1 = lax.dot_general(gx, w1_ref[...], (((1,), (1,)), ((), ())),
                         preferred_element_type=jnp.float32) + b1_ref[...]
    h1 = jnp.where(h1 > 0, h1, jnp.float32(0.01) * h1)
    h2 = lax.dot_general(h1, w2_ref[...], (((1,), (1,)), ((), ())),
                         preferred_element_type=jnp.float32) + b2_ref[...]
    m = jnp.max(h2, axis=1, keepdims=True)
    lse = jnp.log(jnp.sum(jnp.exp(h2 - m), axis=1, keepdims=True))
    o_ref[...] = h2 - m - lse


def _final(colors2, batchp, emb128, w1, b1, w2, b2):
    return pl.pallas_call(
        _final_body,
        in_specs=[
            pl.BlockSpec((ROWS, 128), lambda: (0, 0)),
            pl.BlockSpec((ROWS, 128), lambda: (0, 0)),
            pl.BlockSpec((128, 32), lambda: (0, 0)),
            pl.BlockSpec((256, 32), lambda: (0, 0)),
            pl.BlockSpec((1, 256), lambda: (0, 0)),
            pl.BlockSpec((16, 256), lambda: (0, 0)),
            pl.BlockSpec((1, 16), lambda: (0, 0)),
        ],
        out_specs=pl.BlockSpec((128, 16), lambda: (0, 0)),
        out_shape=jax.ShapeDtypeStruct((128, 16), jnp.float32),
    )(colors2, batchp, emb128, w1, b1, w2, b2)


def kernel(x, edge_index, batch, emb, lin1_w, lin1_b, lin2_w, lin2_b):
    src = jnp.pad(edge_index[0], (0, EPAD - NEDGE)).reshape(EDGE_ROWS, 128)
    dst = jnp.pad(edge_index[1], (0, EPAD - NEDGE),
                  constant_values=N).reshape(EDGE_ROWS, 128)
    zeros = jnp.zeros((NPAD,), jnp.int32)

    h = _argmax_mix(x)
    for step in range(2):
        part = _seg_sum_sc(h.reshape(NPAD), src, dst, zeros)
        valb = _comb(h, part[0].reshape(ROWS, 128), part[1].reshape(ROWS, 128))
        f = _first_occ(valb)
        h = _rank(valb, f, mix_out=(step == 0))

    batchp = jnp.pad(batch, (0, NPAD - N),
                     constant_values=NGRAPH).reshape(ROWS, 128)
    return _final(h, batchp, emb[:NGRAPH], lin1_w,
                  lin1_b.reshape(1, 256), lin2_w, lin2_b.reshape(1, 16))


# speedup vs baseline: 19.6637x; 1.5940x over previous
import functools

import jax
import jax.numpy as jnp
from jax import lax
from jax.experimental import pallas as pl
from jax.experimental.pallas import tpu as pltpu
from jax.experimental.pallas import tpu_sc as plsc

N = 10000
NPAD = 10240
ROWS = NPAD // 128
NGRAPH = 128
NEDGE = 320000

SC_CORES = 2
SC_SUBCORES = 16
SC_WORKERS = SC_CORES * SC_SUBCORES
EDGE_ROWS_PER_W = 80
EDGE_ROWS = SC_WORKERS * EDGE_ROWS_PER_W
EPAD = EDGE_ROWS * 128

MIX_M = 0x45D9F3B
K_NEIGH = 0x9E3779B1 - (1 << 32)
K_POS = 0x85EBCA6B - (1 << 32)
SIGN = -0x80000000
IMAX = 0x7FFFFFFF


def _mix_i32(a):
    m = jnp.int32(MIX_M)
    a = (a ^ lax.shift_right_logical(a, 16)) * m
    a = (a ^ lax.shift_right_logical(a, 16)) * m
    return a ^ lax.shift_right_logical(a, 16)


def _argmax_body(x_ref, o_ref):
    xb = x_ref[...]
    mx = jnp.max(xb, axis=1, keepdims=True)
    it = lax.broadcasted_iota(jnp.int32, xb.shape, 1)
    cand = jnp.where(xb == mx, it, jnp.int32(128))
    idx = jnp.min(cand, axis=1)
    o_ref[...] = _mix_i32(jnp.reshape(idx, (8, 128)))


def _argmax_mix(x):
    return pl.pallas_call(
        _argmax_body,
        grid=(ROWS // 8,),
        in_specs=[pl.BlockSpec((1024, 128), lambda i: (i, 0))],
        out_specs=pl.BlockSpec((8, 128), lambda i: (i, 0)),
        out_shape=jax.ShapeDtypeStruct((ROWS, 128), jnp.int32),
    )(x)


def _seg_body(h_hbm, src_hbm, dst_hbm, zeros_hbm, out_hbm,
              src_v, dst_v, gath_v, acc_sh, h_sh, gsem, ssem):
    cid = lax.axis_index("c")
    sid = lax.axis_index("s")
    wid = cid * SC_SUBCORES + sid
    base = wid * EDGE_ROWS_PER_W
    pltpu.sync_copy(src_hbm.at[pl.ds(base, EDGE_ROWS_PER_W)], src_v)
    pltpu.sync_copy(dst_hbm.at[pl.ds(base, EDGE_ROWS_PER_W)], dst_v)

    @pl.when(sid == 0)
    def _():
        pltpu.sync_copy(zeros_hbm, acc_sh)

    @pl.when(sid == 1)
    def _():
        pltpu.sync_copy(h_hbm, h_sh)

    plsc.subcore_barrier()

    for b in range(8):
        pltpu.async_copy(h_sh.at[src_v.at[b]], gath_v.at[b], gsem)

    def row(t, carry):
        pltpu.make_async_copy(h_sh.at[src_v.at[t]],
                              gath_v.at[t], gsem).wait()
        pltpu.async_copy(gath_v.at[t], acc_sh.at[dst_v.at[t]], ssem,
                         add=True)

        @pl.when(t < EDGE_ROWS_PER_W - 8)
        def _():
            pltpu.async_copy(h_sh.at[src_v.at[t + 8]], gath_v.at[t + 8],
                             gsem)

        return carry

    lax.fori_loop(0, EDGE_ROWS_PER_W, row, 0)
    pltpu.make_async_copy(src_hbm.at[pl.ds(0, EDGE_ROWS_PER_W)],
                          gath_v, ssem).wait()
    plsc.subcore_barrier()

    @pl.when(sid == 0)
    def _():
        pltpu.sync_copy(acc_sh, out_hbm.at[cid])


@functools.cache
def _seg_sum_sc_fn():
    return functools.partial(
        pl.kernel,
        out_type=jax.ShapeDtypeStruct((SC_CORES, NPAD), jnp.int32),
        mesh=plsc.VectorSubcoreMesh(
            core_axis_name="c", subcore_axis_name="s",
            num_cores=SC_CORES, num_subcores=SC_SUBCORES),
        scratch_types=[
            pltpu.VMEM((EDGE_ROWS_PER_W, 128), jnp.int32),
            pltpu.VMEM((EDGE_ROWS_PER_W, 128), jnp.int32),
            pltpu.VMEM((EDGE_ROWS_PER_W, 128), jnp.int32),
            pltpu.VMEM_SHARED((NPAD,), jnp.int32),
            pltpu.VMEM_SHARED((NPAD,), jnp.int32),
            pltpu.SemaphoreType.DMA,
            pltpu.SemaphoreType.DMA,
        ],
    )(_seg_body)


def _seg_sum_sc(h_flat, src, dst, zeros):
    return _seg_sum_sc_fn()(h_flat, src, dst, zeros)


def _comb_body(h_ref, na_ref, nb_ref, o_ref):
    h = h_ref[...]
    neigh = na_ref[...] + nb_ref[...]
    val = _mix_i32(h + jnp.int32(K_NEIGH) * neigh)
    gidx = (lax.broadcasted_iota(jnp.int32, (ROWS, 128), 0) * 128
            + lax.broadcasted_iota(jnp.int32, (ROWS, 128), 1))
    o_ref[...] = jnp.where(gidx >= N, jnp.int32(IMAX), val ^ jnp.int32(SIGN))


def _comb(h2d, na, nb):
    return pl.pallas_call(
        _comb_body,
        in_specs=[pl.BlockSpec((ROWS, 128), lambda: (0, 0))] * 3,
        out_specs=pl.BlockSpec((ROWS, 128), lambda: (0, 0)),
        out_shape=jax.ShapeDtypeStruct((ROWS, 128), jnp.int32),
    )(h2d, na, nb)


def _first_body(vfull_ref, vblk_ref, o_ref):
    pid = pl.program_id(0)
    vb = vblk_ref[...]
    for js in range(8):
        row = vb[js:js + 1, :]
        a = jnp.reshape(row, (128, 1))
        j_row = pid * 8 + js
        jidx = (lax.broadcasted_iota(jnp.int32, (128, 1), 0) + j_row * 128)

        def body(k, acc):
            ch = vfull_ref[pl.ds(k, 1), :]
            kidx = k * 128 + lax.broadcasted_iota(jnp.int32, (1, 128), 1)
            hit = (a == ch) & (kidx < jidx)
            return acc + jnp.where(hit, jnp.int32(1), jnp.int32(0))

        acc = lax.fori_loop(0, j_row + 1, body,
                            jnp.zeros((128, 128), jnp.int32))
        cnt = jnp.reshape(jnp.sum(acc, axis=1), (1, 128))
        o_ref[js:js + 1, :] = jnp.where(cnt == 0, row, jnp.int32(IMAX))


def _first_occ(valb):
    return pl.pallas_call(
        _first_body,
        grid=(ROWS // 8,),
        in_specs=[pl.BlockSpec((ROWS, 128), lambda i: (0, 0)),
                  pl.BlockSpec((8, 128), lambda i: (i, 0))],
        out_specs=pl.BlockSpec((8, 128), lambda i: (i, 0)),
        out_shape=jax.ShapeDtypeStruct((ROWS, 128), jnp.int32),
    )(valb, valb)


def _rank_body(mix_out, fv_ref, vblk_ref, o_ref):
    vb = vblk_ref[...]
    for js in range(8):
        a = jnp.reshape(vb[js:js + 1, :], (128, 1))

        def body(k, acc):
            ch = fv_ref[pl.ds(k, 1), :]
            return acc + jnp.where(ch < a, jnp.int32(1), jnp.int32(0))

        acc = lax.fori_loop(0, ROWS, body, jnp.zeros((128, 128), jnp.int32))
        rank = jnp.sum(acc, axis=1)
        if mix_out:
            rank = _mix_i32(rank)
        o_ref[js:js + 1, :] = jnp.reshape(rank, (1, 128))


def _rank(valb, fv, mix_out):
    return pl.pallas_call(
        functools.partial(_rank_body, mix_out),
        grid=(ROWS // 8,),
        in_specs=[pl.BlockSpec((ROWS, 128), lambda i: (0, 0)),
                  pl.BlockSpec((8, 128), lambda i: (i, 0))],
        out_specs=pl.BlockSpec((8, 128), lambda i: (i, 0)),
        out_shape=jax.ShapeDtypeStruct((ROWS, 128), jnp.int32),
    )(fv, valb)


def _final_body(colors_ref, batch_ref, emb_ref, w1_ref, b1_ref,
                w2_ref, b2_ref, o_ref):
    g_row = lax.broadcasted_iota(jnp.int32, (1, 128), 1)
    s_col = lax.broadcasted_iota(jnp.int32, (128, 1), 0)

    def starts_body(r, acc):
        bt = jnp.reshape(batch_ref[pl.ds(r, 1), :], (128, 1))
        return acc + jnp.where(bt < g_row, jnp.int32(1), jnp.int32(0))

    sacc = lax.fori_loop(0, ROWS, starts_body,
                         jnp.zeros((128, 128), jnp.int32))
    starts_row = jnp.reshape(jnp.sum(sacc, axis=0), (1, 128))

    def gsig_body(r, gacc):
        bt = jnp.reshape(batch_ref[pl.ds(r, 1), :], (128, 1))
        ct = jnp.reshape(colors_ref[pl.ds(r, 1), :], (128, 1))
        eqg = bt == g_row
        st_e = jnp.sum(jnp.where(eqg, starts_row, jnp.int32(0)),
                       axis=1, keepdims=True)
        flat = r * 128 + s_col
        sig = _mix_i32(ct + jnp.int32(K_POS) * (flat - st_e))
        return gacc + jnp.where(eqg, sig, jnp.int32(0))

    gacc = lax.fori_loop(0, ROWS, gsig_body,
                         jnp.zeros((128, 128), jnp.int32))
    gsig_row = jnp.reshape(jnp.sum(gacc, axis=0), (1, 128))

    gb = gsig_row ^ jnp.int32(SIGN)
    gt = jnp.reshape(gb, (128, 1))
    dup = jnp.sum(jnp.where((gt == gb) & (g_row < s_col),
                            jnp.int32(1), jnp.int32(0)), axis=1)
    f_row = jnp.reshape(jnp.where(dup == 0, jnp.int32(1), jnp.int32(0)),
                        (1, 128))
    rank = jnp.sum(jnp.where((gb < gt) & (f_row != 0),
                             jnp.int32(1), jnp.int32(0)),
                   axis=1)
    oh = (jnp.reshape(rank, (128, 1)) == g_row).astype(jnp.float32)
    gx = jnp.dot(oh, emb_ref[...], preferred_element_type=jnp.float32)
    h1 = lax.dot_general(gx, w1_ref[...], (((1,), (1,)), ((), ())),
                         preferred_element_type=jnp.float32) + b1_ref[...]
    h1 = jnp.where(h1 > 0, h1, jnp.float32(0.01) * h1)
    h2 = lax.dot_general(h1, w2_ref[...], (((1,), (1,)), ((), ())),
                         preferred_element_type=jnp.float32) + b2_ref[...]
    m = jnp.max(h2, axis=1, keepdims=True)
    lse = jnp.log(jnp.sum(jnp.exp(h2 - m), axis=1, keepdims=True))
    o_ref[...] = h2 - m - lse


def _final(colors2, batchp, emb128, w1, b1, w2, b2):
    return pl.pallas_call(
        _final_body,
        in_specs=[
            pl.BlockSpec((ROWS, 128), lambda: (0, 0)),
            pl.BlockSpec((ROWS, 128), lambda: (0, 0)),
            pl.BlockSpec((128, 32), lambda: (0, 0)),
            pl.BlockSpec((256, 32), lambda: (0, 0)),
            pl.BlockSpec((1, 256), lambda: (0, 0)),
            pl.BlockSpec((16, 256), lambda: (0, 0)),
            pl.BlockSpec((1, 16), lambda: (0, 0)),
        ],
        out_specs=pl.BlockSpec((128, 16), lambda: (0, 0)),
        out_shape=jax.ShapeDtypeStruct((128, 16), jnp.float32),
    )(colors2, batchp, emb128, w1, b1, w2, b2)


def kernel(x, edge_index, batch, emb, lin1_w, lin1_b, lin2_w, lin2_b):
    src = jnp.pad(edge_index[0], (0, EPAD - NEDGE)).reshape(EDGE_ROWS, 128)
    dst = jnp.pad(edge_index[1], (0, EPAD - NEDGE),
                  constant_values=N).reshape(EDGE_ROWS, 128)
    zeros = jnp.zeros((NPAD,), jnp.int32)

    h = _argmax_mix(x)
    for step in range(2):
        part = _seg_sum_sc(h.reshape(NPAD), src, dst, zeros)
        valb = _comb(h, part[0].reshape(ROWS, 128), part[1].reshape(ROWS, 128))
        f = _first_occ(valb)
        h = _rank(valb, f, mix_out=(step == 0))

    batchp = jnp.pad(batch, (0, NPAD - N),
                     constant_values=NGRAPH).reshape(ROWS, 128)
    return _final(h, batchp, emb[:NGRAPH], lin1_w,
                  lin1_b.reshape(1, 256), lin2_w, lin2_b.reshape(1, 16))


# speedup vs baseline: 34.6241x; 1.7608x over previous
import functools

import jax
import jax.numpy as jnp
from jax import lax
from jax.experimental import pallas as pl
from jax.experimental.pallas import tpu as pltpu
from jax.experimental.pallas import tpu_sc as plsc

N = 10000
NPAD = 10240
ROWS = NPAD // 128
NGRAPH = 128
NEDGE = 320000

SC_CORES = 2
SC_SUBCORES = 16
SC_WORKERS = SC_CORES * SC_SUBCORES
EDGE_ROWS_PER_W = 80
EDGE_ROWS = SC_WORKERS * EDGE_ROWS_PER_W
EPAD = EDGE_ROWS * 128

MIX_M = 0x45D9F3B
K_NEIGH = 0x9E3779B1 - (1 << 32)
K_POS = 0x85EBCA6B - (1 << 32)
SIGN = -0x80000000
IMAX = 0x7FFFFFFF


def _mix_i32(a):
    m = jnp.int32(MIX_M)
    a = (a ^ lax.shift_right_logical(a, 16)) * m
    a = (a ^ lax.shift_right_logical(a, 16)) * m
    return a ^ lax.shift_right_logical(a, 16)


def _argmax_body(x_ref, o_ref):
    xb = x_ref[...]
    mx = jnp.max(xb, axis=1, keepdims=True)
    it = lax.broadcasted_iota(jnp.int32, xb.shape, 1)
    cand = jnp.where(xb == mx, it, jnp.int32(128))
    idx = jnp.min(cand, axis=1)
    o_ref[...] = _mix_i32(jnp.reshape(idx, (8, 128)))


def _argmax_mix(x):
    return pl.pallas_call(
        _argmax_body,
        grid=(ROWS // 8,),
        in_specs=[pl.BlockSpec((1024, 128), lambda i: (i, 0))],
        out_specs=pl.BlockSpec((8, 128), lambda i: (i, 0)),
        out_shape=jax.ShapeDtypeStruct((ROWS, 128), jnp.int32),
    )(x)


def _seg_body(h_hbm, src_hbm, dst_hbm, zeros_hbm, out_hbm,
              src_v, dst_v, gath_v, acc_sh, h_sh, gsem, ssem):
    cid = lax.axis_index("c")
    sid = lax.axis_index("s")
    wid = cid * SC_SUBCORES + sid
    base = wid * EDGE_ROWS_PER_W
    pltpu.sync_copy(src_hbm.at[pl.ds(base, EDGE_ROWS_PER_W)], src_v)
    pltpu.sync_copy(dst_hbm.at[pl.ds(base, EDGE_ROWS_PER_W)], dst_v)

    @pl.when(sid == 0)
    def _():
        pltpu.sync_copy(zeros_hbm, acc_sh)

    @pl.when(sid == 1)
    def _():
        pltpu.sync_copy(h_hbm, h_sh)

    plsc.subcore_barrier()

    for b in range(8):
        pltpu.async_copy(h_sh.at[src_v.at[b]], gath_v.at[b], gsem)

    def row(t, carry):
        pltpu.make_async_copy(h_sh.at[src_v.at[t]],
                              gath_v.at[t], gsem).wait()
        pltpu.async_copy(gath_v.at[t], acc_sh.at[dst_v.at[t]], ssem,
                         add=True)

        @pl.when(t < EDGE_ROWS_PER_W - 8)
        def _():
            pltpu.async_copy(h_sh.at[src_v.at[t + 8]], gath_v.at[t + 8],
                             gsem)

        return carry

    lax.fori_loop(0, EDGE_ROWS_PER_W, row, 0)
    pltpu.make_async_copy(src_hbm.at[pl.ds(0, EDGE_ROWS_PER_W)],
                          gath_v, ssem).wait()
    plsc.subcore_barrier()

    @pl.when(sid == 0)
    def _():
        pltpu.sync_copy(acc_sh, out_hbm.at[cid])


@functools.cache
def _seg_sum_sc_fn():
    return functools.partial(
        pl.kernel,
        out_type=jax.ShapeDtypeStruct((SC_CORES, NPAD), jnp.int32),
        mesh=plsc.VectorSubcoreMesh(
            core_axis_name="c", subcore_axis_name="s",
            num_cores=SC_CORES, num_subcores=SC_SUBCORES),
        scratch_types=[
            pltpu.VMEM((EDGE_ROWS_PER_W, 128), jnp.int32),
            pltpu.VMEM((EDGE_ROWS_PER_W, 128), jnp.int32),
            pltpu.VMEM((EDGE_ROWS_PER_W, 128), jnp.int32),
            pltpu.VMEM_SHARED((NPAD,), jnp.int32),
            pltpu.VMEM_SHARED((NPAD,), jnp.int32),
            pltpu.SemaphoreType.DMA,
            pltpu.SemaphoreType.DMA,
        ],
    )(_seg_body)


def _seg_sum_sc(h_flat, src, dst, zeros):
    return _seg_sum_sc_fn()(h_flat, src, dst, zeros)


def _comb_body(h_ref, na_ref, nb_ref, o_ref):
    h = h_ref[...]
    neigh = na_ref[...] + nb_ref[...]
    val = _mix_i32(h + jnp.int32(K_NEIGH) * neigh)
    gidx = (lax.broadcasted_iota(jnp.int32, (ROWS, 128), 0) * 128
            + lax.broadcasted_iota(jnp.int32, (ROWS, 128), 1))
    o_ref[...] = jnp.where(gidx >= N, jnp.int32(IMAX), val ^ jnp.int32(SIGN))


def _comb(h2d, na, nb):
    return pl.pallas_call(
        _comb_body,
        in_specs=[pl.BlockSpec((ROWS, 128), lambda: (0, 0))] * 3,
        out_specs=pl.BlockSpec((ROWS, 128), lambda: (0, 0)),
        out_shape=jax.ShapeDtypeStruct((ROWS, 128), jnp.int32),
    )(h2d, na, nb)


SORT_N = 16384
SR = SORT_N // 128


def _sort_body(mix_out, key_ref, orank_ref, oidx_ref):
    key80 = key_ref[...]
    key = jnp.concatenate(
        [key80, jnp.full((SR - ROWS, 128), IMAX, jnp.int32)], axis=0)
    ri = lax.broadcasted_iota(jnp.int32, (SR, 128), 0)
    ci = lax.broadcasted_iota(jnp.int32, (SR, 128), 1)
    flat = ri * 128 + ci
    val = flat
    for p in range(1, 15):
        k = 1 << p
        dirmask = (flat & k) == 0
        for q in range(p - 1, -1, -1):
            j = 1 << q
            if j >= 128:
                m, axis, size, bit = j // 128, 0, SR, (ri & (j // 128)) == 0
            else:
                m, axis, size, bit = j, 1, 128, (ci & j) == 0

            def xorshuf(x, m=m, axis=axis, size=size, bit=bit):
                return jnp.where(bit, pltpu.roll(x, size - m, axis),
                                 pltpu.roll(x, m, axis))

            pk, pv = xorshuf(key), xorshuf(val)
            lower = (flat & j) == 0
            cond_min = lower == dirmask
            takep = (cond_min & (pk < key)) | (~cond_min & (pk > key))
            key = jnp.where(takep, pk, key)
            val = jnp.where(takep, pv, val)
    prevk = pltpu.roll(key, 1, 1)
    prev = jnp.where(ci == 0, pltpu.roll(prevk, 1, 0), prevk)
    flag = jnp.where((key != prev) & (flat > 0), jnp.int32(1), jnp.int32(0))
    x = flag
    for d in (1, 2, 4, 8, 16, 32, 64):
        x = x + jnp.where(ci >= d, pltpu.roll(x, d, 1), 0)
    rowtot = jnp.broadcast_to(x[:, 127:128], (SR, 128))
    y = rowtot
    for d in (1, 2, 4, 8, 16, 32, 64):
        y = y + jnp.where(ri >= d, pltpu.roll(y, d, 0), 0)
    rank = x + y - rowtot
    if mix_out:
        rank = _mix_i32(rank)
    orank_ref[...] = rank
    oidx_ref[...] = val


def _sort_rank(valb, mix_out):
    return pl.pallas_call(
        functools.partial(_sort_body, mix_out),
        in_specs=[pl.BlockSpec((ROWS, 128), lambda: (0, 0))],
        out_specs=[pl.BlockSpec((SR, 128), lambda: (0, 0))] * 2,
        out_shape=[jax.ShapeDtypeStruct((SR, 128), jnp.int32)] * 2,
    )(valb)


def _scat_body(rank_hbm, idx_hbm, out_hbm, rank_v, idx_v, sem):
    cid = lax.axis_index("c")
    sid = lax.axis_index("s")
    wid = cid * SC_SUBCORES + sid
    base = wid * (SR // SC_WORKERS)
    pltpu.sync_copy(rank_hbm.at[pl.ds(base, SR // SC_WORKERS)], rank_v)
    pltpu.sync_copy(idx_hbm.at[pl.ds(base, SR // SC_WORKERS)], idx_v)
    for r in range(SR // SC_WORKERS):
        pltpu.async_copy(rank_v.at[r], out_hbm.at[idx_v.at[r]], sem)
    pltpu.make_async_copy(rank_hbm.at[pl.ds(0, SR // SC_WORKERS)], rank_v,
                          sem).wait()


@functools.cache
def _scat_sc_fn():
    return functools.partial(
        pl.kernel,
        out_type=jax.ShapeDtypeStruct((SORT_N,), jnp.int32),
        mesh=plsc.VectorSubcoreMesh(
            core_axis_name="c", subcore_axis_name="s",
            num_cores=SC_CORES, num_subcores=SC_SUBCORES),
        scratch_types=[
            pltpu.VMEM((SR // SC_WORKERS, 128), jnp.int32),
            pltpu.VMEM((SR // SC_WORKERS, 128), jnp.int32),
            pltpu.SemaphoreType.DMA,
        ],
    )(_scat_body)


def _final_body(colors_ref, batch_ref, emb_ref, w1_ref, b1_ref,
                w2_ref, b2_ref, o_ref):
    g_row = lax.broadcasted_iota(jnp.int32, (1, 128), 1)
    s_col = lax.broadcasted_iota(jnp.int32, (128, 1), 0)

    def starts_body(r, acc):
        bt = jnp.reshape(batch_ref[pl.ds(r, 1), :], (128, 1))
        return acc + jnp.where(bt < g_row, jnp.int32(1), jnp.int32(0))

    sacc = lax.fori_loop(0, ROWS, starts_body,
                         jnp.zeros((128, 128), jnp.int32))
    starts_row = jnp.reshape(jnp.sum(sacc, axis=0), (1, 128))

    def gsig_body(r, gacc):
        bt = jnp.reshape(batch_ref[pl.ds(r, 1), :], (128, 1))
        ct = jnp.reshape(colors_ref[pl.ds(r, 1), :], (128, 1))
        eqg = bt == g_row
        st_e = jnp.sum(jnp.where(eqg, starts_row, jnp.int32(0)),
                       axis=1, keepdims=True)
        flat = r * 128 + s_col
        sig = _mix_i32(ct + jnp.int32(K_POS) * (flat - st_e))
        return gacc + jnp.where(eqg, sig, jnp.int32(0))

    gacc = lax.fori_loop(0, ROWS, gsig_body,
                         jnp.zeros((128, 128), jnp.int32))
    gsig_row = jnp.reshape(jnp.sum(gacc, axis=0), (1, 128))

    gb = gsig_row ^ jnp.int32(SIGN)
    gt = jnp.reshape(gb, (128, 1))
    dup = jnp.sum(jnp.where((gt == gb) & (g_row < s_col),
                            jnp.int32(1), jnp.int32(0)), axis=1)
    f_row = jnp.reshape(jnp.where(dup == 0, jnp.int32(1), jnp.int32(0)),
                        (1, 128))
    rank = jnp.sum(jnp.where((gb < gt) & (f_row != 0),
                             jnp.int32(1), jnp.int32(0)),
                   axis=1)
    oh = (jnp.reshape(rank, (128, 1)) == g_row).astype(jnp.float32)
    gx = jnp.dot(oh, emb_ref[...], preferred_element_type=jnp.float32)
    h1 = lax.dot_general(gx, w1_ref[...], (((1,), (1,)), ((), ())),
                         preferred_element_type=jnp.float32) + b1_ref[...]
    h1 = jnp.where(h1 > 0, h1, jnp.float32(0.01) * h1)
    h2 = lax.dot_general(h1, w2_ref[...], (((1,), (1,)), ((), ())),
                         preferred_element_type=jnp.float32) + b2_ref[...]
    m = jnp.max(h2, axis=1, keepdims=True)
    lse = jnp.log(jnp.sum(jnp.exp(h2 - m), axis=1, keepdims=True))
    o_ref[...] = h2 - m - lse


def _final(colors2, batchp, emb128, w1, b1, w2, b2):
    return pl.pallas_call(
        _final_body,
        in_specs=[
            pl.BlockSpec((ROWS, 128), lambda: (0, 0)),
            pl.BlockSpec((ROWS, 128), lambda: (0, 0)),
            pl.BlockSpec((128, 32), lambda: (0, 0)),
            pl.BlockSpec((256, 32), lambda: (0, 0)),
            pl.BlockSpec((1, 256), lambda: (0, 0)),
            pl.BlockSpec((16, 256), lambda: (0, 0)),
            pl.BlockSpec((1, 16), lambda: (0, 0)),
        ],
        out_specs=pl.BlockSpec((128, 16), lambda: (0, 0)),
        out_shape=jax.ShapeDtypeStruct((128, 16), jnp.float32),
    )(colors2, batchp, emb128, w1, b1, w2, b2)


def kernel(x, edge_index, batch, emb, lin1_w, lin1_b, lin2_w, lin2_b):
    src = jnp.pad(edge_index[0], (0, EPAD - NEDGE)).reshape(EDGE_ROWS, 128)
    dst = jnp.pad(edge_index[1], (0, EPAD - NEDGE),
                  constant_values=N).reshape(EDGE_ROWS, 128)
    zeros = jnp.zeros((NPAD,), jnp.int32)

    h = _argmax_mix(x)
    for step in range(2):
        part = _seg_sum_sc(h.reshape(NPAD), src, dst, zeros)
        valb = _comb(h, part[0].reshape(ROWS, 128), part[1].reshape(ROWS, 128))
        rank2d, idx2d = _sort_rank(valb, mix_out=(step == 0))
        flat = _scat_sc_fn()(rank2d, idx2d)
        h = flat[:NPAD].reshape(ROWS, 128)

    batchp = jnp.pad(batch, (0, NPAD - N),
                     constant_values=NGRAPH).reshape(ROWS, 128)
    return _final(h, batchp, emb[:NGRAPH], lin1_w,
                  lin1_b.reshape(1, 256), lin2_w, lin2_b.reshape(1, 16))


# speedup vs baseline: 53.7906x; 1.5536x over previous
import functools

import jax
import jax.numpy as jnp
from jax import lax
from jax.experimental import pallas as pl
from jax.experimental.pallas import tpu as pltpu
from jax.experimental.pallas import tpu_sc as plsc

N = 10000
NPAD = 10240
ROWS = NPAD // 128
NGRAPH = 128
NEDGE = 320000

SC_CORES = 2
SC_SUBCORES = 16
SC_WORKERS = SC_CORES * SC_SUBCORES
EDGE_ROWS_PER_W = 80
EDGE_ROWS = SC_WORKERS * EDGE_ROWS_PER_W
EPAD = EDGE_ROWS * 128

MIX_M = 0x45D9F3B
K_NEIGH = 0x9E3779B1 - (1 << 32)
K_POS = 0x85EBCA6B - (1 << 32)
SIGN = -0x80000000
IMAX = 0x7FFFFFFF


def _mix_i32(a):
    m = jnp.int32(MIX_M)
    a = (a ^ lax.shift_right_logical(a, 16)) * m
    a = (a ^ lax.shift_right_logical(a, 16)) * m
    return a ^ lax.shift_right_logical(a, 16)


def _argmax_body(x_ref, o_ref):
    xb = x_ref[...]
    mx = jnp.max(xb, axis=1, keepdims=True)
    it = lax.broadcasted_iota(jnp.int32, xb.shape, 1)
    cand = jnp.where(xb == mx, it, jnp.int32(128))
    idx = jnp.min(cand, axis=1)
    o_ref[...] = _mix_i32(jnp.reshape(idx, (8, 128)))


def _argmax_mix(x):
    return pl.pallas_call(
        _argmax_body,
        grid=(ROWS // 8,),
        in_specs=[pl.BlockSpec((1024, 128), lambda i: (i, 0))],
        out_specs=pl.BlockSpec((8, 128), lambda i: (i, 0)),
        out_shape=jax.ShapeDtypeStruct((ROWS, 128), jnp.int32),
    )(x)


def _seg_body(ha_hbm, hb_hbm, src_hbm, dst_hbm, zeros_hbm, out_hbm,
              src_v, dst_v, gath_v, ha_v, hb_v, acc_sh, h_sh, gsem, ssem):
    cid = lax.axis_index("c")
    sid = lax.axis_index("s")
    wid = cid * SC_SUBCORES + sid
    base = wid * EDGE_ROWS_PER_W
    pltpu.sync_copy(src_hbm.at[pl.ds(base, EDGE_ROWS_PER_W)], src_v)
    pltpu.sync_copy(dst_hbm.at[pl.ds(base, EDGE_ROWS_PER_W)], dst_v)

    @pl.when(sid == 0)
    def _():
        pltpu.sync_copy(zeros_hbm.at[pl.ds(0, NPAD)], acc_sh)

    @pl.when(sid == 1)
    def _():
        pltpu.sync_copy(ha_hbm, ha_v)
        pltpu.sync_copy(hb_hbm, hb_v)

        def merge(i, carry):
            ha_v[pl.ds(i * 16, 16)] = (ha_v[pl.ds(i * 16, 16)]
                                       + hb_v[pl.ds(i * 16, 16)]
                                       - jnp.int32(1))
            return carry

        lax.fori_loop(0, NPAD // 16, merge, 0)
        pltpu.sync_copy(ha_v, h_sh)

    plsc.subcore_barrier()

    for b in range(8):
        pltpu.async_copy(h_sh.at[src_v.at[b]], gath_v.at[b], gsem)

    def row(t, carry):
        pltpu.make_async_copy(h_sh.at[src_v.at[t]],
                              gath_v.at[t], gsem).wait()
        pltpu.async_copy(gath_v.at[t], acc_sh.at[dst_v.at[t]], ssem,
                         add=True)

        @pl.when(t < EDGE_ROWS_PER_W - 8)
        def _():
            pltpu.async_copy(h_sh.at[src_v.at[t + 8]], gath_v.at[t + 8],
                             gsem)

        return carry

    lax.fori_loop(0, EDGE_ROWS_PER_W, row, 0)
    pltpu.make_async_copy(src_hbm.at[pl.ds(0, EDGE_ROWS_PER_W)],
                          gath_v, ssem).wait()
    plsc.subcore_barrier()

    @pl.when(sid == 0)
    def _():
        pltpu.sync_copy(acc_sh, out_hbm.at[cid])


@functools.cache
def _seg_sum_sc_fn():
    return functools.partial(
        pl.kernel,
        out_type=jax.ShapeDtypeStruct((SC_CORES, NPAD), jnp.int32),
        mesh=plsc.VectorSubcoreMesh(
            core_axis_name="c", subcore_axis_name="s",
            num_cores=SC_CORES, num_subcores=SC_SUBCORES),
        scratch_types=[
            pltpu.VMEM((EDGE_ROWS_PER_W, 128), jnp.int32),
            pltpu.VMEM((EDGE_ROWS_PER_W, 128), jnp.int32),
            pltpu.VMEM((EDGE_ROWS_PER_W, 128), jnp.int32),
            pltpu.VMEM((NPAD,), jnp.int32),
            pltpu.VMEM((NPAD,), jnp.int32),
            pltpu.VMEM_SHARED((NPAD,), jnp.int32),
            pltpu.VMEM_SHARED((NPAD,), jnp.int32),
            pltpu.SemaphoreType.DMA,
            pltpu.SemaphoreType.DMA,
        ],
    )(_seg_body)


def _seg_sum_sc(ha_flat, hb_flat, src, dst, zeros):
    return _seg_sum_sc_fn()(ha_flat, hb_flat, src, dst, zeros)


def _comb_body(ha_ref, hb_ref, na_ref, nb_ref, o_ref):
    h = ha_ref[...] + hb_ref[...] - jnp.int32(1)
    neigh = na_ref[...] + nb_ref[...]
    val = _mix_i32(h + jnp.int32(K_NEIGH) * neigh)
    gidx = (lax.broadcasted_iota(jnp.int32, (ROWS, 128), 0) * 128
            + lax.broadcasted_iota(jnp.int32, (ROWS, 128), 1))
    o_ref[...] = jnp.where(gidx >= N, jnp.int32(IMAX), val ^ jnp.int32(SIGN))


def _comb(ha2d, hb2d, na, nb):
    return pl.pallas_call(
        _comb_body,
        in_specs=[pl.BlockSpec((ROWS, 128), lambda: (0, 0))] * 4,
        out_specs=pl.BlockSpec((ROWS, 128), lambda: (0, 0)),
        out_shape=jax.ShapeDtypeStruct((ROWS, 128), jnp.int32),
    )(ha2d, hb2d, na, nb)


SORT_N = 16384
SR = SORT_N // 128


def _sort_body(mix_out, key_ref, orank_ref, oidx_ref):
    key80 = key_ref[...]
    key = jnp.concatenate(
        [key80, jnp.full((SR - ROWS, 128), IMAX, jnp.int32)], axis=0)
    ri = lax.broadcasted_iota(jnp.int32, (SR, 128), 0)
    ci = lax.broadcasted_iota(jnp.int32, (SR, 128), 1)
    flat = ri * 128 + ci
    val = flat
    for p in range(1, 15):
        k = 1 << p
        dirmask = (flat & k) == 0
        for q in range(p - 1, -1, -1):
            j = 1 << q
            if j >= 128:
                m, axis, size, bit = j // 128, 0, SR, (ri & (j // 128)) == 0
            else:
                m, axis, size, bit = j, 1, 128, (ci & j) == 0

            def xorshuf(x, m=m, axis=axis, size=size, bit=bit):
                return jnp.where(bit, pltpu.roll(x, size - m, axis),
                                 pltpu.roll(x, m, axis))

            pk, pv = xorshuf(key), xorshuf(val)
            lower = (flat & j) == 0
            cond_min = lower == dirmask
            takep = (cond_min & (pk < key)) | (~cond_min & (pk > key))
            key = jnp.where(takep, pk, key)
            val = jnp.where(takep, pv, val)
    prevk = pltpu.roll(key, 1, 1)
    prev = jnp.where(ci == 0, pltpu.roll(prevk, 1, 0), prevk)
    flag = jnp.where((key != prev) & (flat > 0), jnp.int32(1), jnp.int32(0))
    x = flag
    for d in (1, 2, 4, 8, 16, 32, 64):
        x = x + jnp.where(ci >= d, pltpu.roll(x, d, 1), 0)
    rowtot = jnp.broadcast_to(x[:, 127:128], (SR, 128))
    y = rowtot
    for d in (1, 2, 4, 8, 16, 32, 64):
        y = y + jnp.where(ri >= d, pltpu.roll(y, d, 0), 0)
    rank = x + y - rowtot
    if mix_out:
        rank = _mix_i32(rank)
    orank_ref[...] = rank + jnp.int32(1)
    oidx_ref[...] = val


def _sort_rank(valb, mix_out):
    return pl.pallas_call(
        functools.partial(_sort_body, mix_out),
        in_specs=[pl.BlockSpec((ROWS, 128), lambda: (0, 0))],
        out_specs=[pl.BlockSpec((SR, 128), lambda: (0, 0))] * 2,
        out_shape=[jax.ShapeDtypeStruct((SR, 128), jnp.int32)] * 2,
    )(valb)


def _scat_body(rank_hbm, idx_hbm, zeros_hbm, out_hbm, rank_v, idx_v,
               acc_sh, sem):
    cid = lax.axis_index("c")
    sid = lax.axis_index("s")
    wid = cid * SC_SUBCORES + sid
    rpw = SR // SC_WORKERS
    base = wid * rpw
    pltpu.sync_copy(rank_hbm.at[pl.ds(base, rpw)], rank_v)
    pltpu.sync_copy(idx_hbm.at[pl.ds(base, rpw)], idx_v)

    @pl.when(sid == 0)
    def _():
        pltpu.sync_copy(zeros_hbm, acc_sh)

    plsc.subcore_barrier()
    for r in range(rpw):
        pltpu.async_copy(rank_v.at[r], acc_sh.at[idx_v.at[r]], sem)
    pltpu.make_async_copy(rank_hbm.at[pl.ds(0, rpw)], rank_v, sem).wait()
    plsc.subcore_barrier()

    @pl.when(sid == 0)
    def _():
        pltpu.sync_copy(acc_sh, out_hbm.at[cid])


@functools.cache
def _scat_sc_fn():
    return functools.partial(
        pl.kernel,
        out_type=jax.ShapeDtypeStruct((SC_CORES, SORT_N), jnp.int32),
        mesh=plsc.VectorSubcoreMesh(
            core_axis_name="c", subcore_axis_name="s",
            num_cores=SC_CORES, num_subcores=SC_SUBCORES),
        scratch_types=[
            pltpu.VMEM((SR // SC_WORKERS, 128), jnp.int32),
            pltpu.VMEM((SR // SC_WORKERS, 128), jnp.int32),
            pltpu.VMEM_SHARED((SORT_N,), jnp.int32),
            pltpu.SemaphoreType.DMA,
        ],
    )(_scat_body)


def _final_body(ca_ref, cb_ref, batch_ref, emb_ref, w1_ref, b1_ref,
                w2_ref, b2_ref, o_ref):
    g_row = lax.broadcasted_iota(jnp.int32, (1, 128), 1)
    s_col = lax.broadcasted_iota(jnp.int32, (128, 1), 0)

    def starts_body(r, acc):
        bt = jnp.reshape(batch_ref[pl.ds(r, 1), :], (128, 1))
        return acc + jnp.where(bt < g_row, jnp.int32(1), jnp.int32(0))

    sacc = lax.fori_loop(0, ROWS, starts_body,
                         jnp.zeros((128, 128), jnp.int32))
    starts_row = jnp.reshape(jnp.sum(sacc, axis=0), (1, 128))

    def gsig_body(r, gacc):
        bt = jnp.reshape(batch_ref[pl.ds(r, 1), :], (128, 1))
        ct = jnp.reshape(ca_ref[pl.ds(r, 1), :]
                         + cb_ref[pl.ds(r, 1), :] - jnp.int32(1), (128, 1))
        eqg = bt == g_row
        st_e = jnp.sum(jnp.where(eqg, starts_row, jnp.int32(0)),
                       axis=1, keepdims=True)
        flat = r * 128 + s_col
        sig = _mix_i32(ct + jnp.int32(K_POS) * (flat - st_e))
        return gacc + jnp.where(eqg, sig, jnp.int32(0))

    gacc = lax.fori_loop(0, ROWS, gsig_body,
                         jnp.zeros((128, 128), jnp.int32))
    gsig_row = jnp.reshape(jnp.sum(gacc, axis=0), (1, 128))

    gb = gsig_row ^ jnp.int32(SIGN)
    gt = jnp.reshape(gb, (128, 1))
    dup = jnp.sum(jnp.where((gt == gb) & (g_row < s_col),
                            jnp.int32(1), jnp.int32(0)), axis=1)
    f_row = jnp.reshape(jnp.where(dup == 0, jnp.int32(1), jnp.int32(0)),
                        (1, 128))
    rank = jnp.sum(jnp.where((gb < gt) & (f_row != 0),
                             jnp.int32(1), jnp.int32(0)),
                   axis=1)
    oh = (jnp.reshape(rank, (128, 1)) == g_row).astype(jnp.float32)
    gx = jnp.dot(oh, emb_ref[...], preferred_element_type=jnp.float32)
    h1 = lax.dot_general(gx, w1_ref[...], (((1,), (1,)), ((), ())),
                         preferred_element_type=jnp.float32) + b1_ref[...]
    h1 = jnp.where(h1 > 0, h1, jnp.float32(0.01) * h1)
    h2 = lax.dot_general(h1, w2_ref[...], (((1,), (1,)), ((), ())),
                         preferred_element_type=jnp.float32) + b2_ref[...]
    m = jnp.max(h2, axis=1, keepdims=True)
    lse = jnp.log(jnp.sum(jnp.exp(h2 - m), axis=1, keepdims=True))
    o_ref[...] = h2 - m - lse


def _final(ca, cb, batchp, emb128, w1, b1, w2, b2):
    return pl.pallas_call(
        _final_body,
        in_specs=[
            pl.BlockSpec((ROWS, 128), lambda: (0, 0)),
            pl.BlockSpec((ROWS, 128), lambda: (0, 0)),
            pl.BlockSpec((ROWS, 128), lambda: (0, 0)),
            pl.BlockSpec((128, 32), lambda: (0, 0)),
            pl.BlockSpec((256, 32), lambda: (0, 0)),
            pl.BlockSpec((1, 256), lambda: (0, 0)),
            pl.BlockSpec((16, 256), lambda: (0, 0)),
            pl.BlockSpec((1, 16), lambda: (0, 0)),
        ],
        out_specs=pl.BlockSpec((128, 16), lambda: (0, 0)),
        out_shape=jax.ShapeDtypeStruct((128, 16), jnp.float32),
    )(ca, cb, batchp, emb128, w1, b1, w2, b2)


def kernel(x, edge_index, batch, emb, lin1_w, lin1_b, lin2_w, lin2_b):
    src = jnp.pad(edge_index[0], (0, EPAD - NEDGE)).reshape(EDGE_ROWS, 128)
    dst = jnp.pad(edge_index[1], (0, EPAD - NEDGE),
                  constant_values=N).reshape(EDGE_ROWS, 128)
    zeros = jnp.zeros((SORT_N,), jnp.int32)
    ones = jnp.ones((NPAD,), jnp.int32)

    ha = _argmax_mix(x).reshape(NPAD)
    hb = ones
    for step in range(2):
        part = _seg_sum_sc(ha, hb, src, dst, zeros)
        valb = _comb(ha.reshape(ROWS, 128), hb.reshape(ROWS, 128),
                     part[0].reshape(ROWS, 128), part[1].reshape(ROWS, 128))
        rank2d, idx2d = _sort_rank(valb, mix_out=(step == 0))
        scat = _scat_sc_fn()(rank2d, idx2d, zeros)
        ha = scat[0, :NPAD]
        hb = scat[1, :NPAD]

    batchp = jnp.pad(batch, (0, NPAD - N),
                     constant_values=NGRAPH).reshape(ROWS, 128)
    return _final(ha.reshape(ROWS, 128), hb.reshape(ROWS, 128), batchp,
                  emb[:NGRAPH], lin1_w,
                  lin1_b.reshape(1, 256), lin2_w, lin2_b.reshape(1, 16))


# speedup vs baseline: 55.0394x; 1.0232x over previous
import functools

import jax
import jax.numpy as jnp
from jax import lax
from jax.experimental import pallas as pl
from jax.experimental.pallas import tpu as pltpu
from jax.experimental.pallas import tpu_sc as plsc

N = 10000
NPAD = 10240
ROWS = NPAD // 128
NGRAPH = 128
NEDGE = 320000

SC_CORES = 2
SC_SUBCORES = 16
SC_WORKERS = SC_CORES * SC_SUBCORES
EDGE_ROWS_PER_W = 80
EDGE_ROWS = SC_WORKERS * EDGE_ROWS_PER_W
EPAD = EDGE_ROWS * 128

MIX_M = 0x45D9F3B
K_NEIGH = 0x9E3779B1 - (1 << 32)
K_POS = 0x85EBCA6B - (1 << 32)
SIGN = -0x80000000
IMAX = 0x7FFFFFFF


def _mix_i32(a):
    m = jnp.int32(MIX_M)
    a = (a ^ lax.shift_right_logical(a, 16)) * m
    a = (a ^ lax.shift_right_logical(a, 16)) * m
    return a ^ lax.shift_right_logical(a, 16)


def _argmax_body(x_ref, o_ref):
    xb = x_ref[...]
    mx = jnp.max(xb, axis=1, keepdims=True)
    it = lax.broadcasted_iota(jnp.int32, xb.shape, 1)
    cand = jnp.where(xb == mx, it, jnp.int32(128))
    idx = jnp.min(cand, axis=1)
    o_ref[...] = _mix_i32(jnp.reshape(idx, (8, 128)))


def _argmax_mix(x):
    return pl.pallas_call(
        _argmax_body,
        grid=(ROWS // 8,),
        in_specs=[pl.BlockSpec((1024, 128), lambda i: (i, 0))],
        out_specs=pl.BlockSpec((8, 128), lambda i: (i, 0)),
        out_shape=jax.ShapeDtypeStruct((ROWS, 128), jnp.int32),
    )(x)


def _seg_body(ha_hbm, hb_hbm, src_hbm, dst_hbm, zeros_hbm, out_hbm,
              src_v, dst_v, gath_v, ha_v, hb_v, acc_sh, h_sh, gsem, ssem):
    cid = lax.axis_index("c")
    sid = lax.axis_index("s")
    wid = cid * SC_SUBCORES + sid
    base = wid * EDGE_ROWS_PER_W
    pltpu.sync_copy(src_hbm.at[pl.ds(base, EDGE_ROWS_PER_W)], src_v)
    pltpu.sync_copy(dst_hbm.at[pl.ds(base, EDGE_ROWS_PER_W)], dst_v)

    @pl.when(sid == 0)
    def _():
        pltpu.sync_copy(zeros_hbm.at[pl.ds(0, NPAD)], acc_sh)

    @pl.when(sid == 1)
    def _():
        pltpu.sync_copy(ha_hbm, ha_v)
        pltpu.sync_copy(hb_hbm, hb_v)

        def merge(i, carry):
            ha_v[pl.ds(i * 16, 16)] = (ha_v[pl.ds(i * 16, 16)]
                                       + hb_v[pl.ds(i * 16, 16)]
                                       - jnp.int32(1))
            return carry

        lax.fori_loop(0, NPAD // 16, merge, 0)
        pltpu.sync_copy(ha_v, h_sh)

    plsc.subcore_barrier()

    for b in range(8):
        pltpu.async_copy(h_sh.at[src_v.at[b]], gath_v.at[b], gsem)

    def row(t, carry):
        pltpu.make_async_copy(h_sh.at[src_v.at[t]],
                              gath_v.at[t], gsem).wait()
        pltpu.async_copy(gath_v.at[t], acc_sh.at[dst_v.at[t]], ssem,
                         add=True)

        @pl.when(t < EDGE_ROWS_PER_W - 8)
        def _():
            pltpu.async_copy(h_sh.at[src_v.at[t + 8]], gath_v.at[t + 8],
                             gsem)

        return carry

    lax.fori_loop(0, EDGE_ROWS_PER_W, row, 0)
    pltpu.make_async_copy(src_hbm.at[pl.ds(0, EDGE_ROWS_PER_W)],
                          gath_v, ssem).wait()
    plsc.subcore_barrier()

    @pl.when(sid == 0)
    def _():
        pltpu.sync_copy(acc_sh, out_hbm.at[cid])


@functools.cache
def _seg_sum_sc_fn():
    return functools.partial(
        pl.kernel,
        out_type=jax.ShapeDtypeStruct((SC_CORES, NPAD), jnp.int32),
        mesh=plsc.VectorSubcoreMesh(
            core_axis_name="c", subcore_axis_name="s",
            num_cores=SC_CORES, num_subcores=SC_SUBCORES),
        scratch_types=[
            pltpu.VMEM((EDGE_ROWS_PER_W, 128), jnp.int32),
            pltpu.VMEM((EDGE_ROWS_PER_W, 128), jnp.int32),
            pltpu.VMEM((EDGE_ROWS_PER_W, 128), jnp.int32),
            pltpu.VMEM((NPAD,), jnp.int32),
            pltpu.VMEM((NPAD,), jnp.int32),
            pltpu.VMEM_SHARED((NPAD,), jnp.int32),
            pltpu.VMEM_SHARED((NPAD,), jnp.int32),
            pltpu.SemaphoreType.DMA,
            pltpu.SemaphoreType.DMA,
        ],
    )(_seg_body)


def _seg_sum_sc(ha_flat, hb_flat, src, dst, zeros):
    return _seg_sum_sc_fn()(ha_flat, hb_flat, src, dst, zeros)


SORT_N = 16384
SR = SORT_N // 128


def _sort_body(mix_out, ha_ref, hb_ref, na_ref, nb_ref, orank_ref, oidx_ref):
    h = ha_ref[...] + hb_ref[...] - jnp.int32(1)
    neigh = na_ref[...] + nb_ref[...]
    cv = _mix_i32(h + jnp.int32(K_NEIGH) * neigh)
    gidx = (lax.broadcasted_iota(jnp.int32, (ROWS, 128), 0) * 128
            + lax.broadcasted_iota(jnp.int32, (ROWS, 128), 1))
    key80 = jnp.where(gidx >= N, jnp.int32(IMAX), cv ^ jnp.int32(SIGN))
    key = jnp.concatenate(
        [key80, jnp.full((SR - ROWS, 128), IMAX, jnp.int32)], axis=0)
    ri = lax.broadcasted_iota(jnp.int32, (SR, 128), 0)
    ci = lax.broadcasted_iota(jnp.int32, (SR, 128), 1)
    flat = ri * 128 + ci
    val = flat
    for p in range(1, 15):
        k = 1 << p
        dirmask = (flat & k) == 0
        for q in range(p - 1, -1, -1):
            j = 1 << q
            if j >= 128:
                m, axis, size, bit = j // 128, 0, SR, (ri & (j // 128)) == 0
            else:
                m, axis, size, bit = j, 1, 128, (ci & j) == 0

            def xorshuf(x, m=m, axis=axis, size=size, bit=bit):
                return jnp.where(bit, pltpu.roll(x, size - m, axis),
                                 pltpu.roll(x, m, axis))

            pk, pv = xorshuf(key), xorshuf(val)
            lower = (flat & j) == 0
            cond_min = lower == dirmask
            takep = (cond_min & (pk < key)) | (~cond_min & (pk > key))
            key = jnp.where(takep, pk, key)
            val = jnp.where(takep, pv, val)
    prevk = pltpu.roll(key, 1, 1)
    prev = jnp.where(ci == 0, pltpu.roll(prevk, 1, 0), prevk)
    flag = jnp.where((key != prev) & (flat > 0), jnp.int32(1), jnp.int32(0))
    x = flag
    for d in (1, 2, 4, 8, 16, 32, 64):
        x = x + jnp.where(ci >= d, pltpu.roll(x, d, 1), 0)
    rowtot = jnp.broadcast_to(x[:, 127:128], (SR, 128))
    y = rowtot
    for d in (1, 2, 4, 8, 16, 32, 64):
        y = y + jnp.where(ri >= d, pltpu.roll(y, d, 0), 0)
    rank = x + y - rowtot
    if mix_out:
        rank = _mix_i32(rank)
    orank_ref[...] = rank + jnp.int32(1)
    oidx_ref[...] = val


def _sort_rank(ha2d, hb2d, na, nb, mix_out):
    return pl.pallas_call(
        functools.partial(_sort_body, mix_out),
        in_specs=[pl.BlockSpec((ROWS, 128), lambda: (0, 0))] * 4,
        out_specs=[pl.BlockSpec((SR, 128), lambda: (0, 0))] * 2,
        out_shape=[jax.ShapeDtypeStruct((SR, 128), jnp.int32)] * 2,
    )(ha2d, hb2d, na, nb)


def _scat_body(rank_hbm, idx_hbm, zeros_hbm, out_hbm, rank_v, idx_v,
               acc_sh, sem):
    cid = lax.axis_index("c")
    sid = lax.axis_index("s")
    wid = cid * SC_SUBCORES + sid
    rpw = SR // SC_WORKERS
    base = wid * rpw
    pltpu.sync_copy(rank_hbm.at[pl.ds(base, rpw)], rank_v)
    pltpu.sync_copy(idx_hbm.at[pl.ds(base, rpw)], idx_v)

    @pl.when(sid == 0)
    def _():
        pltpu.sync_copy(zeros_hbm, acc_sh)

    plsc.subcore_barrier()
    for r in range(rpw):
        pltpu.async_copy(rank_v.at[r], acc_sh.at[idx_v.at[r]], sem)
    pltpu.make_async_copy(rank_hbm.at[pl.ds(0, rpw)], rank_v, sem).wait()
    plsc.subcore_barrier()

    @pl.when(sid == 0)
    def _():
        pltpu.sync_copy(acc_sh, out_hbm.at[cid])


@functools.cache
def _scat_sc_fn():
    return functools.partial(
        pl.kernel,
        out_type=jax.ShapeDtypeStruct((SC_CORES, SORT_N), jnp.int32),
        mesh=plsc.VectorSubcoreMesh(
            core_axis_name="c", subcore_axis_name="s",
            num_cores=SC_CORES, num_subcores=SC_SUBCORES),
        scratch_types=[
            pltpu.VMEM((SR // SC_WORKERS, 128), jnp.int32),
            pltpu.VMEM((SR // SC_WORKERS, 128), jnp.int32),
            pltpu.VMEM_SHARED((SORT_N,), jnp.int32),
            pltpu.SemaphoreType.DMA,
        ],
    )(_scat_body)


def _final_body(ca_ref, cb_ref, batch_ref, emb_ref, w1_ref, b1_ref,
                w2_ref, b2_ref, o_ref):
    g_row = lax.broadcasted_iota(jnp.int32, (1, 128), 1)
    s_col = lax.broadcasted_iota(jnp.int32, (128, 1), 0)

    def starts_body(r, acc):
        bt = jnp.reshape(batch_ref[pl.ds(r, 1), :], (128, 1))
        return acc + jnp.where(bt < g_row, jnp.int32(1), jnp.int32(0))

    sacc = lax.fori_loop(0, ROWS, starts_body,
                         jnp.zeros((128, 128), jnp.int32))
    starts_row = jnp.reshape(jnp.sum(sacc, axis=0), (1, 128))

    def gsig_body(r, gacc):
        bt = jnp.reshape(batch_ref[pl.ds(r, 1), :], (128, 1))
        ct = jnp.reshape(ca_ref[pl.ds(r, 1), :]
                         + cb_ref[pl.ds(r, 1), :] - jnp.int32(1), (128, 1))
        eqg = bt == g_row
        st_e = jnp.sum(jnp.where(eqg, starts_row, jnp.int32(0)),
                       axis=1, keepdims=True)
        flat = r * 128 + s_col
        sig = _mix_i32(ct + jnp.int32(K_POS) * (flat - st_e))
        return gacc + jnp.where(eqg, sig, jnp.int32(0))

    gacc = lax.fori_loop(0, ROWS, gsig_body,
                         jnp.zeros((128, 128), jnp.int32))
    gsig_row = jnp.reshape(jnp.sum(gacc, axis=0), (1, 128))

    gb = gsig_row ^ jnp.int32(SIGN)
    gt = jnp.reshape(gb, (128, 1))
    dup = jnp.sum(jnp.where((gt == gb) & (g_row < s_col),
                            jnp.int32(1), jnp.int32(0)), axis=1)
    f_row = jnp.reshape(jnp.where(dup == 0, jnp.int32(1), jnp.int32(0)),
                        (1, 128))
    rank = jnp.sum(jnp.where((gb < gt) & (f_row != 0),
                             jnp.int32(1), jnp.int32(0)),
                   axis=1)
    oh = (jnp.reshape(rank, (128, 1)) == g_row).astype(jnp.float32)
    gx = jnp.dot(oh, emb_ref[...], preferred_element_type=jnp.float32)
    h1 = lax.dot_general(gx, w1_ref[...], (((1,), (1,)), ((), ())),
                         preferred_element_type=jnp.float32) + b1_ref[...]
    h1 = jnp.where(h1 > 0, h1, jnp.float32(0.01) * h1)
    h2 = lax.dot_general(h1, w2_ref[...], (((1,), (1,)), ((), ())),
                         preferred_element_type=jnp.float32) + b2_ref[...]
    m = jnp.max(h2, axis=1, keepdims=True)
    lse = jnp.log(jnp.sum(jnp.exp(h2 - m), axis=1, keepdims=True))
    o_ref[...] = h2 - m - lse


def _final(ca, cb, batchp, emb128, w1, b1, w2, b2):
    return pl.pallas_call(
        _final_body,
        in_specs=[
            pl.BlockSpec((ROWS, 128), lambda: (0, 0)),
            pl.BlockSpec((ROWS, 128), lambda: (0, 0)),
            pl.BlockSpec((ROWS, 128), lambda: (0, 0)),
            pl.BlockSpec((128, 32), lambda: (0, 0)),
            pl.BlockSpec((256, 32), lambda: (0, 0)),
            pl.BlockSpec((1, 256), lambda: (0, 0)),
            pl.BlockSpec((16, 256), lambda: (0, 0)),
            pl.BlockSpec((1, 16), lambda: (0, 0)),
        ],
        out_specs=pl.BlockSpec((128, 16), lambda: (0, 0)),
        out_shape=jax.ShapeDtypeStruct((128, 16), jnp.float32),
    )(ca, cb, batchp, emb128, w1, b1, w2, b2)


def kernel(x, edge_index, batch, emb, lin1_w, lin1_b, lin2_w, lin2_b):
    src = jnp.pad(edge_index[0], (0, EPAD - NEDGE)).reshape(EDGE_ROWS, 128)
    dst = jnp.pad(edge_index[1], (0, EPAD - NEDGE),
                  constant_values=N).reshape(EDGE_ROWS, 128)
    zeros = jnp.zeros((SORT_N,), jnp.int32)
    ones = jnp.ones((NPAD,), jnp.int32)

    ha = _argmax_mix(x).reshape(NPAD)
    hb = ones
    for step in range(2):
        part = _seg_sum_sc(ha, hb, src, dst, zeros)
        rank2d, idx2d = _sort_rank(
            ha.reshape(ROWS, 128), hb.reshape(ROWS, 128),
            part[0].reshape(ROWS, 128), part[1].reshape(ROWS, 128),
            mix_out=(step == 0))
        scat = _scat_sc_fn()(rank2d, idx2d, zeros)
        ha = scat[0, :NPAD]
        hb = scat[1, :NPAD]

    batchp = jnp.pad(batch, (0, NPAD - N),
                     constant_values=NGRAPH).reshape(ROWS, 128)
    return _final(ha.reshape(ROWS, 128), hb.reshape(ROWS, 128), batchp,
                  emb[:NGRAPH], lin1_w,
                  lin1_b.reshape(1, 256), lin2_w, lin2_b.reshape(1, 16))


# speedup vs baseline: 56.7772x; 1.0316x over previous
import functools

import jax
import jax.numpy as jnp
from jax import lax
from jax.experimental import pallas as pl
from jax.experimental.pallas import tpu as pltpu
from jax.experimental.pallas import tpu_sc as plsc

N = 10000
NPAD = 10240
ROWS = NPAD // 128
NGRAPH = 128
NEDGE = 320000

SC_CORES = 2
SC_SUBCORES = 16
SC_WORKERS = SC_CORES * SC_SUBCORES
EDGE_ROWS_PER_W = 80
EDGE_ROWS = SC_WORKERS * EDGE_ROWS_PER_W
EPAD = EDGE_ROWS * 128

MIX_M = 0x45D9F3B
K_NEIGH = 0x9E3779B1 - (1 << 32)
K_POS = 0x85EBCA6B - (1 << 32)
SIGN = -0x80000000
IMAX = 0x7FFFFFFF


def _mix_i32(a):
    m = jnp.int32(MIX_M)
    a = (a ^ lax.shift_right_logical(a, 16)) * m
    a = (a ^ lax.shift_right_logical(a, 16)) * m
    return a ^ lax.shift_right_logical(a, 16)


def _argmax_body(x_ref, o_ref):
    xb = x_ref[...]
    mx = jnp.max(xb, axis=1, keepdims=True)
    it = lax.broadcasted_iota(jnp.int32, xb.shape, 1)
    cand = jnp.where(xb == mx, it, jnp.int32(128))
    idx = jnp.min(cand, axis=1)
    o_ref[...] = _mix_i32(jnp.reshape(idx, (8, 128)))


def _argmax_mix(x):
    return pl.pallas_call(
        _argmax_body,
        grid=(ROWS // 8,),
        in_specs=[pl.BlockSpec((1024, 128), lambda i: (i, 0))],
        out_specs=pl.BlockSpec((8, 128), lambda i: (i, 0)),
        out_shape=jax.ShapeDtypeStruct((ROWS, 128), jnp.int32),
    )(x)


def _seg_body(ha_hbm, hb_hbm, src_hbm, dst_hbm, zeros_hbm, out_hbm,
              src_v, dst_v, gath_v, ha_v, hb_v, acc_sh, h_sh, gsem, ssem):
    cid = lax.axis_index("c")
    sid = lax.axis_index("s")
    wid = cid * SC_SUBCORES + sid
    base = wid * EDGE_ROWS_PER_W
    pltpu.sync_copy(src_hbm.at[pl.ds(base, EDGE_ROWS_PER_W)], src_v)
    pltpu.sync_copy(dst_hbm.at[pl.ds(base, EDGE_ROWS_PER_W)], dst_v)

    @pl.when(sid == 0)
    def _():
        pltpu.sync_copy(zeros_hbm.at[pl.ds(0, NPAD)], acc_sh)

    @pl.when(sid == 1)
    def _():
        pltpu.sync_copy(ha_hbm, ha_v)
        pltpu.sync_copy(hb_hbm, hb_v)

        def merge(i, carry):
            ha_v[pl.ds(i * 16, 16)] = (ha_v[pl.ds(i * 16, 16)]
                                       + hb_v[pl.ds(i * 16, 16)]
                                       - jnp.int32(1))
            return carry

        lax.fori_loop(0, NPAD // 16, merge, 0)
        pltpu.sync_copy(ha_v, h_sh)

    plsc.subcore_barrier()

    for b in range(8):
        pltpu.async_copy(h_sh.at[src_v.at[b]], gath_v.at[b], gsem)

    def row(t, carry):
        pltpu.make_async_copy(h_sh.at[src_v.at[t]],
                              gath_v.at[t], gsem).wait()
        pltpu.async_copy(gath_v.at[t], acc_sh.at[dst_v.at[t]], ssem,
                         add=True)

        @pl.when(t < EDGE_ROWS_PER_W - 8)
        def _():
            pltpu.async_copy(h_sh.at[src_v.at[t + 8]], gath_v.at[t + 8],
                             gsem)

        return carry

    lax.fori_loop(0, EDGE_ROWS_PER_W, row, 0)
    pltpu.make_async_copy(src_hbm.at[pl.ds(0, EDGE_ROWS_PER_W)],
                          gath_v, ssem).wait()
    plsc.subcore_barrier()

    @pl.when(sid == 0)
    def _():
        pltpu.sync_copy(acc_sh, out_hbm.at[cid])


@functools.cache
def _seg_sum_sc_fn():
    return functools.partial(
        pl.kernel,
        out_type=jax.ShapeDtypeStruct((SC_CORES, NPAD), jnp.int32),
        mesh=plsc.VectorSubcoreMesh(
            core_axis_name="c", subcore_axis_name="s",
            num_cores=SC_CORES, num_subcores=SC_SUBCORES),
        scratch_types=[
            pltpu.VMEM((EDGE_ROWS_PER_W, 128), jnp.int32),
            pltpu.VMEM((EDGE_ROWS_PER_W, 128), jnp.int32),
            pltpu.VMEM((EDGE_ROWS_PER_W, 128), jnp.int32),
            pltpu.VMEM((NPAD,), jnp.int32),
            pltpu.VMEM((NPAD,), jnp.int32),
            pltpu.VMEM_SHARED((NPAD,), jnp.int32),
            pltpu.VMEM_SHARED((NPAD,), jnp.int32),
            pltpu.SemaphoreType.DMA,
            pltpu.SemaphoreType.DMA,
        ],
    )(_seg_body)


def _seg_sum_sc(ha_flat, hb_flat, src, dst, zeros):
    return _seg_sum_sc_fn()(ha_flat, hb_flat, src, dst, zeros)


SORT_N = 16384
SR = SORT_N // 128


def _sort_body(mix_out, ha_ref, hb_ref, na_ref, nb_ref, orank_ref, oidx_ref):
    h = ha_ref[...] + hb_ref[...] - jnp.int32(1)
    neigh = na_ref[...] + nb_ref[...]
    cv = _mix_i32(h + jnp.int32(K_NEIGH) * neigh)
    gidx = (lax.broadcasted_iota(jnp.int32, (ROWS, 128), 0) * 128
            + lax.broadcasted_iota(jnp.int32, (ROWS, 128), 1))
    key80 = jnp.where(gidx >= N, jnp.int32(IMAX), cv ^ jnp.int32(SIGN))
    key = jnp.concatenate(
        [key80, jnp.full((SR - ROWS, 128), IMAX, jnp.int32)], axis=0)
    ri = lax.broadcasted_iota(jnp.int32, (SR, 128), 0)
    ci = lax.broadcasted_iota(jnp.int32, (SR, 128), 1)
    flat = ri * 128 + ci
    val = flat
    for p in range(1, 15):
        k = 1 << p
        dirmask = (flat & k) == 0
        for q in range(p - 1, -1, -1):
            j = 1 << q
            if j >= 128:
                m, axis, size, bit = j // 128, 0, SR, (ri & (j // 128)) == 0
            else:
                m, axis, size, bit = j, 1, 128, (ci & j) == 0

            def xorshuf(x, m=m, axis=axis, size=size, bit=bit):
                return jnp.where(bit, pltpu.roll(x, size - m, axis),
                                 pltpu.roll(x, m, axis))

            pk, pv = xorshuf(key), xorshuf(val)
            cond_min = ((flat & j) == 0) == dirmask
            newkey = jnp.where(cond_min, jnp.minimum(key, pk),
                               jnp.maximum(key, pk))
            val = jnp.where(newkey != key, pv, val)
            key = newkey
    prevk = pltpu.roll(key, 1, 1)
    prev = jnp.where(ci == 0, pltpu.roll(prevk, 1, 0), prevk)
    flag = jnp.where((key != prev) & (flat > 0), jnp.int32(1), jnp.int32(0))
    x = flag
    for d in (1, 2, 4, 8, 16, 32, 64):
        x = x + jnp.where(ci >= d, pltpu.roll(x, d, 1), 0)
    rowtot = jnp.broadcast_to(x[:, 127:128], (SR, 128))
    y = rowtot
    for d in (1, 2, 4, 8, 16, 32, 64):
        y = y + jnp.where(ri >= d, pltpu.roll(y, d, 0), 0)
    rank = x + y - rowtot
    if mix_out:
        rank = _mix_i32(rank)
    orank_ref[...] = rank + jnp.int32(1)
    oidx_ref[...] = val


def _sort_rank(ha2d, hb2d, na, nb, mix_out):
    return pl.pallas_call(
        functools.partial(_sort_body, mix_out),
        in_specs=[pl.BlockSpec((ROWS, 128), lambda: (0, 0))] * 4,
        out_specs=[pl.BlockSpec((SR, 128), lambda: (0, 0))] * 2,
        out_shape=[jax.ShapeDtypeStruct((SR, 128), jnp.int32)] * 2,
    )(ha2d, hb2d, na, nb)


def _scat_body(rank_hbm, idx_hbm, zeros_hbm, out_hbm, rank_v, idx_v,
               acc_sh, sem):
    cid = lax.axis_index("c")
    sid = lax.axis_index("s")
    wid = cid * SC_SUBCORES + sid
    rpw = SR // SC_WORKERS
    base = wid * rpw
    pltpu.sync_copy(rank_hbm.at[pl.ds(base, rpw)], rank_v)
    pltpu.sync_copy(idx_hbm.at[pl.ds(base, rpw)], idx_v)

    @pl.when(sid == 0)
    def _():
        pltpu.sync_copy(zeros_hbm, acc_sh)

    plsc.subcore_barrier()
    for r in range(rpw):
        pltpu.async_copy(rank_v.at[r], acc_sh.at[idx_v.at[r]], sem)
    pltpu.make_async_copy(rank_hbm.at[pl.ds(0, rpw)], rank_v, sem).wait()
    plsc.subcore_barrier()

    @pl.when(sid == 0)
    def _():
        pltpu.sync_copy(acc_sh, out_hbm.at[cid])


@functools.cache
def _scat_sc_fn():
    return functools.partial(
        pl.kernel,
        out_type=jax.ShapeDtypeStruct((SC_CORES, SORT_N), jnp.int32),
        mesh=plsc.VectorSubcoreMesh(
            core_axis_name="c", subcore_axis_name="s",
            num_cores=SC_CORES, num_subcores=SC_SUBCORES),
        scratch_types=[
            pltpu.VMEM((SR // SC_WORKERS, 128), jnp.int32),
            pltpu.VMEM((SR // SC_WORKERS, 128), jnp.int32),
            pltpu.VMEM_SHARED((SORT_N,), jnp.int32),
            pltpu.SemaphoreType.DMA,
        ],
    )(_scat_body)


def _final_body(ca_ref, cb_ref, batch_ref, emb_ref, w1_ref, b1_ref,
                w2_ref, b2_ref, o_ref):
    g_row = lax.broadcasted_iota(jnp.int32, (1, 128), 1)
    s_col = lax.broadcasted_iota(jnp.int32, (128, 1), 0)

    def starts_body(r, acc):
        bt = jnp.reshape(batch_ref[pl.ds(r, 1), :], (128, 1))
        return acc + jnp.where(bt < g_row, jnp.int32(1), jnp.int32(0))

    sacc = lax.fori_loop(0, ROWS, starts_body,
                         jnp.zeros((128, 128), jnp.int32))
    starts_row = jnp.reshape(jnp.sum(sacc, axis=0), (1, 128))

    def gsig_body(r, gacc):
        bt = jnp.reshape(batch_ref[pl.ds(r, 1), :], (128, 1))
        ct = jnp.reshape(ca_ref[pl.ds(r, 1), :]
                         + cb_ref[pl.ds(r, 1), :] - jnp.int32(1), (128, 1))
        eqg = bt == g_row
        st_e = jnp.sum(jnp.where(eqg, starts_row, jnp.int32(0)),
                       axis=1, keepdims=True)
        flat = r * 128 + s_col
        sig = _mix_i32(ct + jnp.int32(K_POS) * (flat - st_e))
        return gacc + jnp.where(eqg, sig, jnp.int32(0))

    gacc = lax.fori_loop(0, ROWS, gsig_body,
                         jnp.zeros((128, 128), jnp.int32))
    gsig_row = jnp.reshape(jnp.sum(gacc, axis=0), (1, 128))

    gb = gsig_row ^ jnp.int32(SIGN)
    gt = jnp.reshape(gb, (128, 1))
    dup = jnp.sum(jnp.where((gt == gb) & (g_row < s_col),
                            jnp.int32(1), jnp.int32(0)), axis=1)
    f_row = jnp.reshape(jnp.where(dup == 0, jnp.int32(1), jnp.int32(0)),
                        (1, 128))
    rank = jnp.sum(jnp.where((gb < gt) & (f_row != 0),
                             jnp.int32(1), jnp.int32(0)),
                   axis=1)
    oh = (jnp.reshape(rank, (128, 1)) == g_row).astype(jnp.float32)
    gx = jnp.dot(oh, emb_ref[...], preferred_element_type=jnp.float32)
    h1 = lax.dot_general(gx, w1_ref[...], (((1,), (1,)), ((), ())),
                         preferred_element_type=jnp.float32) + b1_ref[...]
    h1 = jnp.where(h1 > 0, h1, jnp.float32(0.01) * h1)
    h2 = lax.dot_general(h1, w2_ref[...], (((1,), (1,)), ((), ())),
                         preferred_element_type=jnp.float32) + b2_ref[...]
    m = jnp.max(h2, axis=1, keepdims=True)
    lse = jnp.log(jnp.sum(jnp.exp(h2 - m), axis=1, keepdims=True))
    o_ref[...] = h2 - m - lse


def _final(ca, cb, batchp, emb128, w1, b1, w2, b2):
    return pl.pallas_call(
        _final_body,
        in_specs=[
            pl.BlockSpec((ROWS, 128), lambda: (0, 0)),
            pl.BlockSpec((ROWS, 128), lambda: (0, 0)),
            pl.BlockSpec((ROWS, 128), lambda: (0, 0)),
            pl.BlockSpec((128, 32), lambda: (0, 0)),
            pl.BlockSpec((256, 32), lambda: (0, 0)),
            pl.BlockSpec((1, 256), lambda: (0, 0)),
            pl.BlockSpec((16, 256), lambda: (0, 0)),
            pl.BlockSpec((1, 16), lambda: (0, 0)),
        ],
        out_specs=pl.BlockSpec((128, 16), lambda: (0, 0)),
        out_shape=jax.ShapeDtypeStruct((128, 16), jnp.float32),
    )(ca, cb, batchp, emb128, w1, b1, w2, b2)


def kernel(x, edge_index, batch, emb, lin1_w, lin1_b, lin2_w, lin2_b):
    src = jnp.pad(edge_index[0], (0, EPAD - NEDGE)).reshape(EDGE_ROWS, 128)
    dst = jnp.pad(edge_index[1], (0, EPAD - NEDGE),
                  constant_values=N).reshape(EDGE_ROWS, 128)
    zeros = jnp.zeros((SORT_N,), jnp.int32)
    ones = jnp.ones((NPAD,), jnp.int32)

    ha = _argmax_mix(x).reshape(NPAD)
    hb = ones
    for step in range(2):
        part = _seg_sum_sc(ha, hb, src, dst, zeros)
        rank2d, idx2d = _sort_rank(
            ha.reshape(ROWS, 128), hb.reshape(ROWS, 128),
            part[0].reshape(ROWS, 128), part[1].reshape(ROWS, 128),
            mix_out=(step == 0))
        scat = _scat_sc_fn()(rank2d, idx2d, zeros)
        ha = scat[0, :NPAD]
        hb = scat[1, :NPAD]

    batchp = jnp.pad(batch, (0, NPAD - N),
                     constant_values=NGRAPH).reshape(ROWS, 128)
    return _final(ha.reshape(ROWS, 128), hb.reshape(ROWS, 128), batchp,
                  emb[:NGRAPH], lin1_w,
                  lin1_b.reshape(1, 256), lin2_w, lin2_b.reshape(1, 16))


# speedup vs baseline: 59.5878x; 1.0495x over previous
import functools

import jax
import jax.numpy as jnp
from jax import lax
from jax.experimental import pallas as pl
from jax.experimental.pallas import tpu as pltpu
from jax.experimental.pallas import tpu_sc as plsc

N = 10000
NPAD = 10240
ROWS = NPAD // 128
NGRAPH = 128
NEDGE = 320000

SC_CORES = 2
SC_SUBCORES = 16
SC_WORKERS = SC_CORES * SC_SUBCORES
EDGE_ROWS_PER_W = 80
EDGE_ROWS = SC_WORKERS * EDGE_ROWS_PER_W
EPAD = EDGE_ROWS * 128

MIX_M = 0x45D9F3B
K_NEIGH = 0x9E3779B1 - (1 << 32)
K_POS = 0x85EBCA6B - (1 << 32)
SIGN = -0x80000000
IMAX = 0x7FFFFFFF


def _mix_i32(a):
    m = jnp.int32(MIX_M)
    a = (a ^ lax.shift_right_logical(a, 16)) * m
    a = (a ^ lax.shift_right_logical(a, 16)) * m
    return a ^ lax.shift_right_logical(a, 16)


def _argmax_body(x_ref, o_ref):
    xb = x_ref[...]
    mx = jnp.max(xb, axis=1, keepdims=True)
    it = lax.broadcasted_iota(jnp.int32, xb.shape, 1)
    cand = jnp.where(xb == mx, it, jnp.int32(128))
    idx = jnp.min(cand, axis=1)
    o_ref[...] = _mix_i32(jnp.reshape(idx, (8, 128)))


def _argmax_mix(x):
    return pl.pallas_call(
        _argmax_body,
        grid=(ROWS // 8,),
        in_specs=[pl.BlockSpec((1024, 128), lambda i: (i, 0))],
        out_specs=pl.BlockSpec((8, 128), lambda i: (i, 0)),
        out_shape=jax.ShapeDtypeStruct((ROWS, 128), jnp.int32),
    )(x)


def _seg_body(h_hbm, src_hbm, dst_hbm, zeros_hbm, out_hbm,
              src_v, dst_v, gath_v, acc_sh, h_sh, gsem, ssem):
    cid = lax.axis_index("c")
    sid = lax.axis_index("s")
    wid = cid * SC_SUBCORES + sid
    base = wid * EDGE_ROWS_PER_W
    pltpu.sync_copy(src_hbm.at[pl.ds(base, EDGE_ROWS_PER_W)], src_v)
    pltpu.sync_copy(dst_hbm.at[pl.ds(base, EDGE_ROWS_PER_W)], dst_v)

    @pl.when(sid == 0)
    def _():
        pltpu.sync_copy(zeros_hbm, acc_sh)

    @pl.when(sid == 1)
    def _():
        pltpu.sync_copy(h_hbm, h_sh)

    plsc.subcore_barrier()

    for b in range(8):
        pltpu.async_copy(h_sh.at[src_v.at[b]], gath_v.at[b], gsem)

    def row(t, carry):
        pltpu.make_async_copy(h_sh.at[src_v.at[t]],
                              gath_v.at[t], gsem).wait()
        pltpu.async_copy(gath_v.at[t], acc_sh.at[dst_v.at[t]], ssem,
                         add=True)

        @pl.when(t < EDGE_ROWS_PER_W - 8)
        def _():
            pltpu.async_copy(h_sh.at[src_v.at[t + 8]], gath_v.at[t + 8],
                             gsem)

        return carry

    lax.fori_loop(0, EDGE_ROWS_PER_W, row, 0)
    pltpu.make_async_copy(src_hbm.at[pl.ds(0, EDGE_ROWS_PER_W)],
                          gath_v, ssem).wait()
    plsc.subcore_barrier()

    @pl.when(sid == 0)
    def _():
        pltpu.sync_copy(acc_sh, out_hbm.at[cid])


@functools.cache
def _seg_sum_sc_fn():
    return functools.partial(
        pl.kernel,
        out_type=jax.ShapeDtypeStruct((SC_CORES, NPAD), jnp.int32),
        mesh=plsc.VectorSubcoreMesh(
            core_axis_name="c", subcore_axis_name="s",
            num_cores=SC_CORES, num_subcores=SC_SUBCORES),
        scratch_types=[
            pltpu.VMEM((EDGE_ROWS_PER_W, 128), jnp.int32),
            pltpu.VMEM((EDGE_ROWS_PER_W, 128), jnp.int32),
            pltpu.VMEM((EDGE_ROWS_PER_W, 128), jnp.int32),
            pltpu.VMEM_SHARED((NPAD,), jnp.int32),
            pltpu.VMEM_SHARED((NPAD,), jnp.int32),
            pltpu.SemaphoreType.DMA,
            pltpu.SemaphoreType.DMA,
        ],
    )(_seg_body)


def _seg_sum_sc(h_flat, src, dst, zeros):
    return _seg_sum_sc_fn()(h_flat, src, dst, zeros)


SORT_N = 16384
SR = SORT_N // 128


def _bitonic(key, val, ri, ci, flat):
    for p in range(1, 15):
        k = 1 << p
        dirmask = (flat & k) == 0
        for q in range(p - 1, -1, -1):
            j = 1 << q
            if j >= 128:
                m, axis, size, bit = j // 128, 0, SR, (ri & (j // 128)) == 0
            else:
                m, axis, size, bit = j, 1, 128, (ci & j) == 0

            def xorshuf(x, m=m, axis=axis, size=size, bit=bit):
                return jnp.where(bit, pltpu.roll(x, size - m, axis),
                                 pltpu.roll(x, m, axis))

            pk, pv = xorshuf(key), xorshuf(val)
            cond_min = ((flat & j) == 0) == dirmask
            newkey = jnp.where(cond_min, jnp.minimum(key, pk),
                               jnp.maximum(key, pk))
            val = jnp.where(newkey != key, pv, val)
            key = newkey
    return key, val


def _sort_body(mix_out, h_ref, na_ref, nb_ref, o_ref):
    h = h_ref[...]
    neigh = na_ref[...] + nb_ref[...]
    cv = _mix_i32(h + jnp.int32(K_NEIGH) * neigh)
    gidx = (lax.broadcasted_iota(jnp.int32, (ROWS, 128), 0) * 128
            + lax.broadcasted_iota(jnp.int32, (ROWS, 128), 1))
    key80 = jnp.where(gidx >= N, jnp.int32(IMAX), cv ^ jnp.int32(SIGN))
    key = jnp.concatenate(
        [key80, jnp.full((SR - ROWS, 128), IMAX, jnp.int32)], axis=0)
    ri = lax.broadcasted_iota(jnp.int32, (SR, 128), 0)
    ci = lax.broadcasted_iota(jnp.int32, (SR, 128), 1)
    flat = ri * 128 + ci
    key, val = _bitonic(key, flat, ri, ci, flat)
    prevk = pltpu.roll(key, 1, 1)
    prev = jnp.where(ci == 0, pltpu.roll(prevk, 1, 0), prevk)
    flag = jnp.where((key != prev) & (flat > 0), jnp.int32(1), jnp.int32(0))
    x = flag
    for d in (1, 2, 4, 8, 16, 32, 64):
        x = x + jnp.where(ci >= d, pltpu.roll(x, d, 1), 0)
    rowtot = jnp.broadcast_to(x[:, 127:128], (SR, 128))
    y = rowtot
    for d in (1, 2, 4, 8, 16, 32, 64):
        y = y + jnp.where(ri >= d, pltpu.roll(y, d, 0), 0)
    rank = x + y - rowtot
    if mix_out:
        rank = _mix_i32(rank)
    _, back = _bitonic(val, rank, ri, ci, flat)
    o_ref[...] = back[:ROWS]


def _sort_rank(h2d, na, nb, mix_out):
    return pl.pallas_call(
        functools.partial(_sort_body, mix_out),
        in_specs=[pl.BlockSpec((ROWS, 128), lambda: (0, 0))] * 3,
        out_specs=pl.BlockSpec((ROWS, 128), lambda: (0, 0)),
        out_shape=jax.ShapeDtypeStruct((ROWS, 128), jnp.int32),
    )(h2d, na, nb)


def _final_body(colors_ref, batch_ref, emb_ref, w1_ref, b1_ref,
                w2_ref, b2_ref, o_ref):
    g_row = lax.broadcasted_iota(jnp.int32, (1, 128), 1)
    s_col = lax.broadcasted_iota(jnp.int32, (128, 1), 0)

    def starts_body(r, acc):
        bt = jnp.reshape(batch_ref[pl.ds(r, 1), :], (128, 1))
        return acc + jnp.where(bt < g_row, jnp.int32(1), jnp.int32(0))

    sacc = lax.fori_loop(0, ROWS, starts_body,
                         jnp.zeros((128, 128), jnp.int32))
    starts_row = jnp.reshape(jnp.sum(sacc, axis=0), (1, 128))

    def gsig_body(r, gacc):
        bt = jnp.reshape(batch_ref[pl.ds(r, 1), :], (128, 1))
        ct = jnp.reshape(colors_ref[pl.ds(r, 1), :], (128, 1))
        eqg = bt == g_row
        st_e = jnp.sum(jnp.where(eqg, starts_row, jnp.int32(0)),
                       axis=1, keepdims=True)
        flat = r * 128 + s_col
        sig = _mix_i32(ct + jnp.int32(K_POS) * (flat - st_e))
        return gacc + jnp.where(eqg, sig, jnp.int32(0))

    gacc = lax.fori_loop(0, ROWS, gsig_body,
                         jnp.zeros((128, 128), jnp.int32))
    gsig_row = jnp.reshape(jnp.sum(gacc, axis=0), (1, 128))

    gb = gsig_row ^ jnp.int32(SIGN)
    gt = jnp.reshape(gb, (128, 1))
    dup = jnp.sum(jnp.where((gt == gb) & (g_row < s_col),
                            jnp.int32(1), jnp.int32(0)), axis=1)
    f_row = jnp.reshape(jnp.where(dup == 0, jnp.int32(1), jnp.int32(0)),
                        (1, 128))
    rank = jnp.sum(jnp.where((gb < gt) & (f_row != 0),
                             jnp.int32(1), jnp.int32(0)),
                   axis=1)
    oh = (jnp.reshape(rank, (128, 1)) == g_row).astype(jnp.float32)
    gx = jnp.dot(oh, emb_ref[...], preferred_element_type=jnp.float32)
    h1 = lax.dot_general(gx, w1_ref[...], (((1,), (1,)), ((), ())),
                         preferred_element_type=jnp.float32) + b1_ref[...]
    h1 = jnp.where(h1 > 0, h1, jnp.float32(0.01) * h1)
    h2 = lax.dot_general(h1, w2_ref[...], (((1,), (1,)), ((), ())),
                         preferred_element_type=jnp.float32) + b2_ref[...]
    m = jnp.max(h2, axis=1, keepdims=True)
    lse = jnp.log(jnp.sum(jnp.exp(h2 - m), axis=1, keepdims=True))
    o_ref[...] = h2 - m - lse


def _final(colors2, batchp, emb128, w1, b1, w2, b2):
    return pl.pallas_call(
        _final_body,
        in_specs=[
            pl.BlockSpec((ROWS, 128), lambda: (0, 0)),
            pl.BlockSpec((ROWS, 128), lambda: (0, 0)),
            pl.BlockSpec((128, 32), lambda: (0, 0)),
            pl.BlockSpec((256, 32), lambda: (0, 0)),
            pl.BlockSpec((1, 256), lambda: (0, 0)),
            pl.BlockSpec((16, 256), lambda: (0, 0)),
            pl.BlockSpec((1, 16), lambda: (0, 0)),
        ],
        out_specs=pl.BlockSpec((128, 16), lambda: (0, 0)),
        out_shape=jax.ShapeDtypeStruct((128, 16), jnp.float32),
    )(colors2, batchp, emb128, w1, b1, w2, b2)


def kernel(x, edge_index, batch, emb, lin1_w, lin1_b, lin2_w, lin2_b):
    src = jnp.pad(edge_index[0], (0, EPAD - NEDGE)).reshape(EDGE_ROWS, 128)
    dst = jnp.pad(edge_index[1], (0, EPAD - NEDGE),
                  constant_values=N).reshape(EDGE_ROWS, 128)
    zeros = jnp.zeros((NPAD,), jnp.int32)

    h = _argmax_mix(x)
    for step in range(2):
        part = _seg_sum_sc(h.reshape(NPAD), src, dst, zeros)
        h = _sort_rank(h, part[0].reshape(ROWS, 128),
                       part[1].reshape(ROWS, 128), mix_out=(step == 0))

    batchp = jnp.pad(batch, (0, NPAD - N),
                     constant_values=NGRAPH).reshape(ROWS, 128)
    return _final(h, batchp, emb[:NGRAPH], lin1_w,
                  lin1_b.reshape(1, 256), lin2_w, lin2_b.reshape(1, 16))


# speedup vs baseline: 60.1327x; 1.0091x over previous
import functools

import jax
import jax.numpy as jnp
from jax import lax
from jax.experimental import pallas as pl
from jax.experimental.pallas import tpu as pltpu
from jax.experimental.pallas import tpu_sc as plsc

N = 10000
NPAD = 10240
ROWS = NPAD // 128
NGRAPH = 128
NEDGE = 320000

SC_CORES = 2
SC_SUBCORES = 16
SC_WORKERS = SC_CORES * SC_SUBCORES
EDGE_ROWS_PER_W = 80
EDGE_ROWS = SC_WORKERS * EDGE_ROWS_PER_W
EPAD = EDGE_ROWS * 128

MIX_M = 0x45D9F3B
K_NEIGH = 0x9E3779B1 - (1 << 32)
K_POS = 0x85EBCA6B - (1 << 32)
SIGN = -0x80000000
IMAX = 0x7FFFFFFF


def _mix_i32(a):
    m = jnp.int32(MIX_M)
    a = (a ^ lax.shift_right_logical(a, 16)) * m
    a = (a ^ lax.shift_right_logical(a, 16)) * m
    return a ^ lax.shift_right_logical(a, 16)


def _argmax_body(x_ref, o_ref):
    xb = x_ref[...]
    mx = jnp.max(xb, axis=1, keepdims=True)
    it = lax.broadcasted_iota(jnp.int32, xb.shape, 1)
    cand = jnp.where(xb == mx, it, jnp.int32(128))
    idx = jnp.min(cand, axis=1)
    o_ref[...] = _mix_i32(jnp.reshape(idx, (8, 128)))


def _argmax_mix(x):
    return pl.pallas_call(
        _argmax_body,
        grid=(ROWS // 8,),
        in_specs=[pl.BlockSpec((1024, 128), lambda i: (i, 0))],
        out_specs=pl.BlockSpec((8, 128), lambda i: (i, 0)),
        out_shape=jax.ShapeDtypeStruct((ROWS, 128), jnp.int32),
    )(x)


def _seg_body(h_hbm, src_hbm, dst_hbm, zeros_hbm, out_hbm,
              src_v, dst_v, gath_v, acc_sh, h_sh, gsem, ssem):
    cid = lax.axis_index("c")
    sid = lax.axis_index("s")
    wid = cid * SC_SUBCORES + sid
    base = wid * EDGE_ROWS_PER_W
    pltpu.sync_copy(src_hbm.at[pl.ds(base, EDGE_ROWS_PER_W)], src_v)
    pltpu.sync_copy(dst_hbm.at[pl.ds(base, EDGE_ROWS_PER_W)], dst_v)

    @pl.when(sid == 0)
    def _():
        pltpu.sync_copy(zeros_hbm, acc_sh)

    @pl.when(sid == 1)
    def _():
        pltpu.sync_copy(h_hbm, h_sh)

    plsc.subcore_barrier()

    for b in range(8):
        pltpu.async_copy(h_sh.at[src_v.at[b]], gath_v.at[b], gsem)

    def row(t, carry):
        pltpu.make_async_copy(h_sh.at[src_v.at[t]],
                              gath_v.at[t], gsem).wait()
        pltpu.async_copy(gath_v.at[t], acc_sh.at[dst_v.at[t]], ssem,
                         add=True)

        @pl.when(t < EDGE_ROWS_PER_W - 8)
        def _():
            pltpu.async_copy(h_sh.at[src_v.at[t + 8]], gath_v.at[t + 8],
                             gsem)

        return carry

    lax.fori_loop(0, EDGE_ROWS_PER_W, row, 0)
    pltpu.make_async_copy(src_hbm.at[pl.ds(0, EDGE_ROWS_PER_W)],
                          gath_v, ssem).wait()
    plsc.subcore_barrier()

    @pl.when(sid == 0)
    def _():
        pltpu.sync_copy(acc_sh, out_hbm.at[cid])


@functools.cache
def _seg_sum_sc_fn():
    return functools.partial(
        pl.kernel,
        out_type=jax.ShapeDtypeStruct((SC_CORES, NPAD), jnp.int32),
        mesh=plsc.VectorSubcoreMesh(
            core_axis_name="c", subcore_axis_name="s",
            num_cores=SC_CORES, num_subcores=SC_SUBCORES),
        scratch_types=[
            pltpu.VMEM((EDGE_ROWS_PER_W, 128), jnp.int32),
            pltpu.VMEM((EDGE_ROWS_PER_W, 128), jnp.int32),
            pltpu.VMEM((EDGE_ROWS_PER_W, 128), jnp.int32),
            pltpu.VMEM_SHARED((NPAD,), jnp.int32),
            pltpu.VMEM_SHARED((NPAD,), jnp.int32),
            pltpu.SemaphoreType.DMA,
            pltpu.SemaphoreType.DMA,
        ],
    )(_seg_body)


def _seg_sum_sc(h_flat, src, dst, zeros):
    return _seg_sum_sc_fn()(h_flat, src, dst, zeros)


SORT_N = 16384
SR = SORT_N // 128


def _bitonic(key, val, ri, ci, flat):
    for p in range(1, 15):
        k = 1 << p
        dirmask = (flat & k) == 0
        for q in range(p - 1, -1, -1):
            j = 1 << q
            if j >= 128:
                m, axis, size, bit = j // 128, 0, SR, (ri & (j // 128)) == 0
            else:
                m, axis, size, bit = j, 1, 128, (ci & j) == 0

            def xorshuf(x, m=m, axis=axis, size=size, bit=bit):
                return jnp.where(bit, pltpu.roll(x, size - m, axis),
                                 pltpu.roll(x, m, axis))

            pk, pv = xorshuf(key), xorshuf(val)
            cond_min = ((flat & j) == 0) == dirmask
            newkey = jnp.where(cond_min, jnp.minimum(key, pk),
                               jnp.maximum(key, pk))
            val = jnp.where(newkey != key, pv, val)
            key = newkey
    return key, val


def _sort_body(mix_out, h_ref, na_ref, nb_ref, o_ref):
    h = h_ref[...]
    neigh = na_ref[...] + nb_ref[...]
    cv = _mix_i32(h + jnp.int32(K_NEIGH) * neigh)
    gidx = (lax.broadcasted_iota(jnp.int32, (ROWS, 128), 0) * 128
            + lax.broadcasted_iota(jnp.int32, (ROWS, 128), 1))
    key80 = jnp.where(gidx >= N, jnp.int32(IMAX), cv ^ jnp.int32(SIGN))
    key = jnp.concatenate(
        [key80, jnp.full((SR - ROWS, 128), IMAX, jnp.int32)], axis=0)
    ri = lax.broadcasted_iota(jnp.int32, (SR, 128), 0)
    ci = lax.broadcasted_iota(jnp.int32, (SR, 128), 1)
    flat = ri * 128 + ci
    key, val = _bitonic(key, flat, ri, ci, flat)
    prevk = pltpu.roll(key, 1, 1)
    prev = jnp.where(ci == 0, pltpu.roll(prevk, 1, 0), prevk)
    flag = jnp.where((key != prev) & (flat > 0), jnp.int32(1), jnp.int32(0))
    x = flag
    for d in (1, 2, 4, 8, 16, 32, 64):
        x = x + jnp.where(ci >= d, pltpu.roll(x, d, 1), 0)
    rowtot = jnp.broadcast_to(x[:, 127:128], (SR, 128))
    y = rowtot
    for d in (1, 2, 4, 8, 16, 32, 64):
        y = y + jnp.where(ri >= d, pltpu.roll(y, d, 0), 0)
    rank = x + y - rowtot
    if mix_out:
        rank = _mix_i32(rank)
    _, back = _bitonic(val, rank, ri, ci, flat)
    o_ref[...] = back[:ROWS]


def _sort_rank(h2d, na, nb, mix_out):
    return pl.pallas_call(
        functools.partial(_sort_body, mix_out),
        in_specs=[pl.BlockSpec((ROWS, 128), lambda: (0, 0))] * 3,
        out_specs=pl.BlockSpec((ROWS, 128), lambda: (0, 0)),
        out_shape=jax.ShapeDtypeStruct((ROWS, 128), jnp.int32),
    )(h2d, na, nb)


def _final_body(sort_args, batch_ref, emb_ref, w1_ref, b1_ref,
                w2_ref, b2_ref, o_ref, colors_scr):
    _sort_body(False, *sort_args, colors_scr)
    g_row = lax.broadcasted_iota(jnp.int32, (1, 128), 1)
    s_col = lax.broadcasted_iota(jnp.int32, (128, 1), 0)

    def starts_body(r, acc):
        bt = jnp.reshape(batch_ref[pl.ds(r, 1), :], (128, 1))
        return acc + jnp.where(bt < g_row, jnp.int32(1), jnp.int32(0))

    sacc = lax.fori_loop(0, ROWS, starts_body,
                         jnp.zeros((128, 128), jnp.int32))
    starts_row = jnp.reshape(jnp.sum(sacc, axis=0), (1, 128))

    def gsig_body(r, gacc):
        bt = jnp.reshape(batch_ref[pl.ds(r, 1), :], (128, 1))
        ct = jnp.reshape(colors_scr[pl.ds(r, 1), :], (128, 1))
        eqg = bt == g_row
        st_e = jnp.sum(jnp.where(eqg, starts_row, jnp.int32(0)),
                       axis=1, keepdims=True)
        flat = r * 128 + s_col
        sig = _mix_i32(ct + jnp.int32(K_POS) * (flat - st_e))
        return gacc + jnp.where(eqg, sig, jnp.int32(0))

    gacc = lax.fori_loop(0, ROWS, gsig_body,
                         jnp.zeros((128, 128), jnp.int32))
    gsig_row = jnp.reshape(jnp.sum(gacc, axis=0), (1, 128))

    gb = gsig_row ^ jnp.int32(SIGN)
    gt = jnp.reshape(gb, (128, 1))
    dup = jnp.sum(jnp.where((gt == gb) & (g_row < s_col),
                            jnp.int32(1), jnp.int32(0)), axis=1)
    f_row = jnp.reshape(jnp.where(dup == 0, jnp.int32(1), jnp.int32(0)),
                        (1, 128))
    rank = jnp.sum(jnp.where((gb < gt) & (f_row != 0),
                             jnp.int32(1), jnp.int32(0)),
                   axis=1)
    oh = (jnp.reshape(rank, (128, 1)) == g_row).astype(jnp.float32)
    gx = jnp.dot(oh, emb_ref[...], preferred_element_type=jnp.float32)
    h1 = lax.dot_general(gx, w1_ref[...], (((1,), (1,)), ((), ())),
                         preferred_element_type=jnp.float32) + b1_ref[...]
    h1 = jnp.where(h1 > 0, h1, jnp.float32(0.01) * h1)
    h2 = lax.dot_general(h1, w2_ref[...], (((1,), (1,)), ((), ())),
                         preferred_element_type=jnp.float32) + b2_ref[...]
    m = jnp.max(h2, axis=1, keepdims=True)
    lse = jnp.log(jnp.sum(jnp.exp(h2 - m), axis=1, keepdims=True))
    o_ref[...] = h2 - m - lse


def _final(h2d, na, nb, batchp, emb128, w1, b1, w2, b2):
    def body(h_ref, na_ref, nb_ref, batch_ref, emb_ref, w1_ref, b1_ref,
             w2_ref, b2_ref, o_ref, colors_scr):
        _final_body((h_ref, na_ref, nb_ref), batch_ref, emb_ref, w1_ref,
                    b1_ref, w2_ref, b2_ref, o_ref, colors_scr)

    return pl.pallas_call(
        body,
        in_specs=[
            pl.BlockSpec((ROWS, 128), lambda: (0, 0)),
            pl.BlockSpec((ROWS, 128), lambda: (0, 0)),
            pl.BlockSpec((ROWS, 128), lambda: (0, 0)),
            pl.BlockSpec((ROWS, 128), lambda: (0, 0)),
            pl.BlockSpec((128, 32), lambda: (0, 0)),
            pl.BlockSpec((256, 32), lambda: (0, 0)),
            pl.BlockSpec((1, 256), lambda: (0, 0)),
            pl.BlockSpec((16, 256), lambda: (0, 0)),
            pl.BlockSpec((1, 16), lambda: (0, 0)),
        ],
        out_specs=pl.BlockSpec((128, 16), lambda: (0, 0)),
        out_shape=jax.ShapeDtypeStruct((128, 16), jnp.float32),
        scratch_shapes=[pltpu.VMEM((ROWS, 128), jnp.int32)],
    )(h2d, na, nb, batchp, emb128, w1, b1, w2, b2)


def kernel(x, edge_index, batch, emb, lin1_w, lin1_b, lin2_w, lin2_b):
    src = jnp.pad(edge_index[0], (0, EPAD - NEDGE)).reshape(EDGE_ROWS, 128)
    dst = jnp.pad(edge_index[1], (0, EPAD - NEDGE),
                  constant_values=N).reshape(EDGE_ROWS, 128)
    zeros = jnp.zeros((NPAD,), jnp.int32)

    h = _argmax_mix(x)
    part = _seg_sum_sc(h.reshape(NPAD), src, dst, zeros)
    h = _sort_rank(h, part[0].reshape(ROWS, 128),
                   part[1].reshape(ROWS, 128), mix_out=True)
    part = _seg_sum_sc(h.reshape(NPAD), src, dst, zeros)

    batchp = jnp.pad(batch, (0, NPAD - N),
                     constant_values=NGRAPH).reshape(ROWS, 128)
    return _final(h, part[0].reshape(ROWS, 128), part[1].reshape(ROWS, 128),
                  batchp, emb[:NGRAPH], lin1_w,
                  lin1_b.reshape(1, 256), lin2_w, lin2_b.reshape(1, 16))
